# baseline (speedup 1.0000x reference)
import math

import jax
import jax.numpy as jnp
from jax import lax
from jax.experimental import pallas as pl
from jax.experimental.pallas import tpu as pltpu
from jax.experimental.pallas import tpu_sc as plsc

N = 10000
E = 320000
D = 128
H = 8
DH = 16

NC = 2
NS = 16
NW = NC * NS
EPW = E // NW
EPT = E // NS
CH = 80
NCHUNK = EPW // CH
NCH2 = EPT // CH
GB = 128
NGB = EPW // GB
GREM = EPW - NGB * GB
NZC = 400
NZN = N // NZC
W2 = D // NC

_mesh = plsc.VectorSubcoreMesh(
    core_axis_name="c", subcore_axis_name="s", num_cores=NC, num_subcores=NS)
_sc_params = pltpu.CompilerParams(use_tc_tiling_on_sc=False)


def _ln_block(x, g, b):
  ones = jnp.ones((D, 1), jnp.float32)
  s1 = jnp.dot(x, ones, preferred_element_type=jnp.float32)
  s2 = jnp.dot(x * x, ones, preferred_element_type=jnp.float32)
  m = s1 * (1.0 / D)
  var = s2 * (1.0 / D) - m * m
  return (x - m) * jax.lax.rsqrt(var + 1e-5) * g + b


def _head_sel():
  r = lax.broadcasted_iota(jnp.int32, (D, 16), 0) // DH
  c = lax.broadcasted_iota(jnp.int32, (D, 16), 1)
  return (r == c).astype(jnp.float32)


def _node_pre_body(x_ref, g_ref, b_ref, wq_ref, bq_ref, wk_ref, bk_ref,
                   wv_ref, bv_ref, q_ref, kv_ref):
  x = x_ref[...]
  xn = _ln_block(x, g_ref[...], b_ref[...])
  q = jnp.dot(xn, wq_ref[...], preferred_element_type=jnp.float32) + bq_ref[...]
  k = jnp.dot(xn, wk_ref[...], preferred_element_type=jnp.float32) + bk_ref[...]
  v = jnp.dot(xn, wv_ref[...], preferred_element_type=jnp.float32) + bv_ref[...]
  q_ref[...] = q
  kv_ref[...] = jnp.concatenate([k, v * xn], axis=1)


def _node_pre(x, g, b, wq, bq, wk, bk, wv, bv):
  blk = 1000
  grid = N // blk
  row = pl.BlockSpec((blk, D), lambda i: (i, 0))
  full = pl.BlockSpec((D, D), lambda i: (0, 0))
  vec = pl.BlockSpec((D,), lambda i: (0,))
  return pl.pallas_call(
      _node_pre_body,
      grid=(grid,),
      in_specs=[row, vec, vec, full, vec, full, vec, full, vec],
      out_specs=[row, pl.BlockSpec((blk, 2 * D), lambda i: (i, 0))],
      out_shape=[jax.ShapeDtypeStruct((N, D), jnp.float32),
                 jax.ShapeDtypeStruct((N, 2 * D), jnp.float32)],
  )(x, g, b, wq, bq, wk, bk, wv, bv)


def _edge_ep_body(e_ref, g_ref, b_ref, wes_ref, bes_ref, ep_ref):
  en = _ln_block(e_ref[...], g_ref[...], b_ref[...])
  esh = (jnp.dot(en, wes_ref[...], preferred_element_type=jnp.float32)
         + bes_ref[...])
  ep_ref[...] = jnp.dot(esh * en, _head_sel(),
                        preferred_element_type=jnp.float32)


def _edge_ep(e, g, b, wes, bes):
  blk = 2000
  grid = E // blk
  row = pl.BlockSpec((blk, D), lambda i: (i, 0))
  full = pl.BlockSpec((D, D), lambda i: (0, 0))
  vec = pl.BlockSpec((D,), lambda i: (0,))
  out = pl.BlockSpec((blk, 16), lambda i: (i, 0))
  return pl.pallas_call(
      _edge_ep_body,
      grid=(grid,),
      in_specs=[row, vec, vec, full, vec],
      out_specs=out,
      out_shape=jax.ShapeDtypeStruct((E, 16), jnp.float32),
  )(e, g, b, wes, bes)


def _gather2_body(q_hbm, kv_hbm, src_hbm, dst_hbm,
                  qs_out, kvd_out,
                  si, di, qb0, qb1, kvb0, kvb1, sg, so):
  wid = lax.axis_index("s") * NC + lax.axis_index("c")
  base0 = wid * EPW
  qbs = (qb0, qb1)
  kvbs = (kvb0, kvb1)

  def body(g, _):
    base = base0 + g * (2 * GB)

    @pl.when(g > 0)
    def _():
      for j in range(2):
        pltpu.make_async_copy(qbs[j], qs_out.at[pl.ds(base, GB)], so).wait()
        pltpu.make_async_copy(kvbs[j], kvd_out.at[pl.ds(base, GB)], so).wait()

    pltpu.sync_copy(src_hbm.at[pl.ds(base, 2 * GB)], si)
    pltpu.sync_copy(dst_hbm.at[pl.ds(base, 2 * GB)], di)
    copies = []
    for j in range(2):
      copies.append(
          pltpu.async_copy(q_hbm.at[si.at[pl.ds(j * GB, GB)]], qbs[j], sg))
      copies.append(
          pltpu.async_copy(kv_hbm.at[di.at[pl.ds(j * GB, GB)]], kvbs[j], sg))
    for cp in copies:
      cp.wait()
    for j in range(2):
      pltpu.async_copy(qbs[j], qs_out.at[pl.ds(base + j * GB, GB)], so)
      pltpu.async_copy(kvbs[j], kvd_out.at[pl.ds(base + j * GB, GB)], so)
    return 0

  lax.fori_loop(0, NGB // 2, body, 0)
  for j in range(2):
    pltpu.make_async_copy(qbs[j], qs_out.at[pl.ds(base0, GB)], so).wait()
    pltpu.make_async_copy(kvbs[j], kvd_out.at[pl.ds(base0, GB)], so).wait()

  rbase = base0 + NGB * GB
  pltpu.sync_copy(src_hbm.at[pl.ds(rbase, GREM)], si.at[pl.ds(0, GREM)])
  pltpu.sync_copy(dst_hbm.at[pl.ds(rbase, GREM)], di.at[pl.ds(0, GREM)])
  cq = pltpu.async_copy(q_hbm.at[si.at[pl.ds(0, GREM)]],
                        qb0.at[pl.ds(0, GREM)], sg)
  ckv = pltpu.async_copy(kv_hbm.at[di.at[pl.ds(0, GREM)]],
                         kvb0.at[pl.ds(0, GREM)], sg)
  cq.wait()
  ckv.wait()
  pltpu.sync_copy(qb0.at[pl.ds(0, GREM)], qs_out.at[pl.ds(rbase, GREM)])
  pltpu.sync_copy(kvb0.at[pl.ds(0, GREM)], kvd_out.at[pl.ds(rbase, GREM)])


def _gather2(q, kv, src, dst):
  f = pl.kernel(
      _gather2_body,
      out_type=[jax.ShapeDtypeStruct((E, D), jnp.float32),
                jax.ShapeDtypeStruct((E, 2 * D), jnp.float32)],
      mesh=_mesh,
      compiler_params=_sc_params,
      scratch_types=[
          pltpu.VMEM((2 * GB,), jnp.int32),
          pltpu.VMEM((2 * GB,), jnp.int32),
          pltpu.VMEM((GB, D), jnp.float32),
          pltpu.VMEM((GB, D), jnp.float32),
          pltpu.VMEM((GB, 2 * D), jnp.float32),
          pltpu.VMEM((GB, 2 * D), jnp.float32),
          pltpu.SemaphoreType.DMA,
          pltpu.SemaphoreType.DMA,
      ],
  )
  return f(q, kv, src, dst)


SZ = 100
EG = E // SZ


def _edge_numuw_body(qs_ref, kd_ref, vmd_ref, ep_ref, num_ref, uw_ref):
  blk = qs_ref.shape[0]
  sel = _head_sel()
  qk = jnp.dot(qs_ref[...] * kd_ref[...], sel,
               preferred_element_type=jnp.float32)
  num = jnp.exp((qk + ep_ref[...]) * (1.0 / math.sqrt(DH)))
  num_ref[...] = num.reshape(blk // SZ, SZ, 16)
  nb = jnp.dot(num, sel.T, preferred_element_type=jnp.float32)
  uw_ref[...] = (nb * vmd_ref[...]).reshape(blk // SZ, SZ, D)


def _edge_numuw(qs, kvd, ep):
  blk = 2000
  bg = blk // SZ
  grid = E // blk
  row = pl.BlockSpec((blk, D), lambda i: (i, 0))
  n16 = pl.BlockSpec((blk, 16), lambda i: (i, 0))
  kcol = pl.BlockSpec((blk, D), lambda i: (i, 0))
  vcol = pl.BlockSpec((blk, D), lambda i: (i, 1))
  return pl.pallas_call(
      _edge_numuw_body,
      grid=(grid,),
      in_specs=[row, kcol, vcol, n16],
      out_specs=[pl.BlockSpec((bg, SZ, 16), lambda i: (i, 0, 0)),
                 pl.BlockSpec((bg, SZ, D), lambda i: (i, 0, 0))],
      out_shape=[jax.ShapeDtypeStruct((EG, SZ, 16), jnp.float32),
                 jax.ShapeDtypeStruct((EG, SZ, D), jnp.float32)],
  )(qs, kvd, kvd, ep)


GQ = 10
NSC_T = EPT // SZ
NG_SCAT = NSC_T // GQ
NSC_W = EPW // SZ
NG_GATH = NSC_W // GQ


def _den_body(num3, src2, dens3_out, den_out,
              is0, is1, vs0, vs1, db0, db1, zb, acc, s0, s1, gs0, gs1):
  cid = lax.axis_index("c")
  sid = lax.axis_index("s")
  wid = sid * NC + cid
  isl = (is0, is1)
  vsl = (vs0, vs1)
  dbl = (db0, db1)
  ssem = (s0, s1)
  gsem = (gs0, gs1)

  zb[...] = jnp.zeros(zb.shape, jnp.float32)
  for j in range((NZN + NS - 1) // NS):
    ci = sid + j * NS
    @pl.when(ci < NZN)
    def _():
      pltpu.sync_copy(zb, acc.at[pl.ds(ci * NZC, NZC)])
  plsc.subcore_barrier()

  cbase = sid * NSC_T

  def abody(t, _):
    for j in range(2):
      gg = 2 * t + j
      cb = cbase + gg * GQ
      @pl.when(t > 0)
      def _():
        pltpu.make_async_copy(num3.at[pl.ds(cb, GQ)], vsl[j], ssem[j]).wait()
      pltpu.sync_copy(src2.at[pl.ds(cb, GQ)], isl[j])
      pltpu.sync_copy(num3.at[pl.ds(cb, GQ)], vsl[j])
      for jj in range(GQ):
        pltpu.async_copy(vsl[j].at[jj], acc.at[isl[j].at[jj]], ssem[j],
                         add=True)
    return 0

  lax.fori_loop(0, NG_SCAT // 2, abody, 0)
  for j in range(2):
    pltpu.make_async_copy(num3.at[pl.ds(cbase, GQ)], vsl[j], ssem[j]).wait()
  plsc.subcore_barrier()

  gbase = wid * NSC_W
  pltpu.sync_copy(src2.at[pl.ds(gbase, GQ)], is0)
  for jj in range(GQ):
    pltpu.async_copy(acc.at[is0.at[jj]], db0.at[jj], gs0)

  def gbody(t, _):
    for j in range(2):
      gg = 2 * t + j
      cb = gbase + gg * GQ
      @pl.when(gg + 1 < NG_GATH)
      def _():
        pltpu.sync_copy(src2.at[pl.ds(cb + GQ, GQ)], isl[1 - j])
        for jj in range(GQ):
          pltpu.async_copy(acc.at[isl[1 - j].at[jj]], dbl[1 - j].at[jj],
                           gsem[1 - j])
      pltpu.make_async_copy(num3.at[pl.ds(cb, GQ)], dbl[j], gsem[j]).wait()
      pltpu.sync_copy(dbl[j], dens3_out.at[pl.ds(cb, GQ)])
    return 0

  lax.fori_loop(0, NG_GATH // 2, gbody, 0)

  for j in range((NZN + NS - 1) // NS):
    ci = sid + j * NS
    @pl.when(jnp.logical_and(ci < NZN, (ci % NC) == cid))
    def _():
      pltpu.sync_copy(acc.at[pl.ds(ci * NZC, NZC)],
                      den_out.at[pl.ds(ci * NZC, NZC)])


def _den_kernel(num3, src2):
  f = pl.kernel(
      _den_body,
      out_type=[jax.ShapeDtypeStruct((E // SZ, SZ, 16), jnp.float32),
                jax.ShapeDtypeStruct((N, 16), jnp.float32)],
      mesh=_mesh,
      compiler_params=_sc_params,
      scratch_types=[
          pltpu.VMEM((GQ, SZ), jnp.int32),
          pltpu.VMEM((GQ, SZ), jnp.int32),
          pltpu.VMEM((GQ, SZ, 16), jnp.float32),
          pltpu.VMEM((GQ, SZ, 16), jnp.float32),
          pltpu.VMEM((GQ, SZ, 16), jnp.float32),
          pltpu.VMEM((GQ, SZ, 16), jnp.float32),
          pltpu.VMEM((NZC, 16), jnp.float32),
          pltpu.VMEM_SHARED((N, 16), jnp.float32),
          pltpu.SemaphoreType.DMA,
          pltpu.SemaphoreType.DMA,
          pltpu.SemaphoreType.DMA,
          pltpu.SemaphoreType.DMA,
      ],
  )
  return f(num3, src2)


GQA = 4
NG_AGG = NSC_T // GQA


def _segsum_split_body(vals3, src2, out_hbm, is0, is1, vs0, vs1, zb, acc,
                       s0, s1):
  cid = lax.axis_index("c")
  sid = lax.axis_index("s")
  c0 = cid * W2
  isl = (is0, is1)
  vsl = (vs0, vs1)
  ssem = (s0, s1)

  zb[...] = jnp.zeros(zb.shape, jnp.float32)
  for j in range((NZN + NS - 1) // NS):
    ci = sid + j * NS
    @pl.when(ci < NZN)
    def _():
      pltpu.sync_copy(zb, acc.at[pl.ds(ci * NZC, NZC)])
  plsc.subcore_barrier()

  cbase = sid * NSC_T

  def abody(t, _):
    for j in range(2):
      gg = 2 * t + j
      cb = cbase + gg * GQA
      @pl.when(t > 0)
      def _():
        pltpu.make_async_copy(
            vals3.at[pl.ds(cb, GQA), :, pl.ds(c0, W2)], vsl[j],
            ssem[j]).wait()
      pltpu.sync_copy(src2.at[pl.ds(cb, GQA)], isl[j])
      pltpu.sync_copy(vals3.at[pl.ds(cb, GQA), :, pl.ds(c0, W2)], vsl[j])
      for jj in range(GQA):
        pltpu.async_copy(vsl[j].at[jj], acc.at[isl[j].at[jj]], ssem[j],
                         add=True)
    return 0

  lax.fori_loop(0, NG_AGG // 2, abody, 0)
  for j in range(2):
    pltpu.make_async_copy(
        vals3.at[pl.ds(cbase, GQA), :, pl.ds(c0, W2)], vsl[j],
        ssem[j]).wait()
  plsc.subcore_barrier()
  for j in range((NZN + NS - 1) // NS):
    ci = sid + j * NS
    @pl.when(ci < NZN)
    def _():
      pltpu.sync_copy(acc.at[pl.ds(ci * NZC, NZC)],
                      out_hbm.at[pl.ds(ci * NZC, NZC), pl.ds(c0, W2)])


def _segsum_split(vals3, src2):
  f = pl.kernel(
      _segsum_split_body,
      out_type=jax.ShapeDtypeStruct((N, D), jnp.float32),
      mesh=_mesh,
      compiler_params=_sc_params,
      scratch_types=[
          pltpu.VMEM((GQA, SZ), jnp.int32),
          pltpu.VMEM((GQA, SZ), jnp.int32),
          pltpu.VMEM((GQA, SZ, W2), jnp.float32),
          pltpu.VMEM((GQA, SZ, W2), jnp.float32),
          pltpu.VMEM((NZC, W2), jnp.float32),
          pltpu.VMEM_SHARED((N, W2), jnp.float32),
          pltpu.SemaphoreType.DMA,
          pltpu.SemaphoreType.DMA,
      ],
  )
  return f(vals3, src2)


def _attn_ffn_body(num_ref, den_ref, e_ref, weo_ref, beo_ref,
                   g_ref, b_ref, w1_ref, b1_ref, w2_ref, b2_ref,
                   attn_ref, oe_ref):
  blk = e_ref.shape[0]
  num = num_ref[...].reshape(blk, 16)
  den = den_ref[...].reshape(blk, 16)
  attn16 = num / (den + 1e-12)
  attn = attn16[:, :H]
  attn_ref[...] = attn
  eau = (jnp.dot(attn, weo_ref[...],
                 preferred_element_type=jnp.float32) + beo_ref[...])
  es = e_ref[...] + eau
  x = _ln_block(es, g_ref[...], b_ref[...])
  h1 = jax.nn.relu(
      jnp.dot(x, w1_ref[...], preferred_element_type=jnp.float32) + b1_ref[...])
  ef = (jnp.dot(h1, w2_ref[...], preferred_element_type=jnp.float32)
        + b2_ref[...])
  oe_ref[...] = es + ef


def _attn_ffn(num, den_s, e, weo, beo, g, b, w1, b1, w2, b2):
  blk = 2000
  bg = blk // SZ
  grid = E // blk
  row = pl.BlockSpec((blk, D), lambda i: (i, 0))
  n16 = pl.BlockSpec((bg, SZ, 16), lambda i: (i, 0, 0))
  vec = pl.BlockSpec((D,), lambda i: (0,))
  return pl.pallas_call(
      _attn_ffn_body,
      grid=(grid,),
      in_specs=[n16, n16, row,
                pl.BlockSpec((H, D), lambda i: (0, 0)), vec,
                vec, vec,
                pl.BlockSpec((D, 2 * D), lambda i: (0, 0)),
                pl.BlockSpec((2 * D,), lambda i: (0,)),
                pl.BlockSpec((2 * D, D), lambda i: (0, 0)), vec],
      out_specs=[pl.BlockSpec((blk, H), lambda i: (i, 0)), row],
      out_shape=[jax.ShapeDtypeStruct((E, H), jnp.float32),
                 jax.ShapeDtypeStruct((E, D), jnp.float32)],
  )(num, den_s, e, weo, beo, g, b, w1, b1, w2, b2)


def _node_post_body(u_ref, den_ref, x_ref, wno_ref, bno_ref, g_ref, b_ref,
                    w1_ref, b1_ref, w2_ref, b2_ref, o_ref):
  den_b = jnp.dot(den_ref[...], _head_sel().T,
                  preferred_element_type=jnp.float32)
  agg = u_ref[...] / (den_b + 1e-30)
  nau = (jnp.dot(agg, wno_ref[...], preferred_element_type=jnp.float32)
         + bno_ref[...])
  ns = x_ref[...] + nau
  x = _ln_block(ns, g_ref[...], b_ref[...])
  h1 = jax.nn.relu(
      jnp.dot(x, w1_ref[...], preferred_element_type=jnp.float32) + b1_ref[...])
  nf = (jnp.dot(h1, w2_ref[...], preferred_element_type=jnp.float32)
        + b2_ref[...])
  o_ref[...] = ns + nf


def _node_post(u, den, x, wno, bno, g, b, w1, b1, w2, b2):
  blk = 1000
  grid = N // blk
  row = pl.BlockSpec((blk, D), lambda i: (i, 0))
  full = pl.BlockSpec((D, D), lambda i: (0, 0))
  vec = pl.BlockSpec((D,), lambda i: (0,))
  return pl.pallas_call(
      _node_post_body,
      grid=(grid,),
      in_specs=[row, pl.BlockSpec((blk, 16), lambda i: (i, 0)),
                row, full, vec, vec, vec,
                pl.BlockSpec((D, 2 * D), lambda i: (0, 0)),
                pl.BlockSpec((2 * D,), lambda i: (0,)),
                pl.BlockSpec((2 * D, D), lambda i: (0, 0)), vec],
      out_specs=row,
      out_shape=jax.ShapeDtypeStruct((N, D), jnp.float32),
  )(u, den, x, wno, bno, g, b, w1, b1, w2, b2)


@jax.jit
def kernel(node_states, edge_index, edge_states, params):
  p = params
  src = edge_index[0]
  dst = edge_index[1]

  q, kv = _node_pre(node_states, p['nln1_g'], p['nln1_b'],
                    p['wq'], p['bq'], p['wk'], p['bk'], p['wv'], p['bv'])

  ep = _edge_ep(edge_states, p['eln1_g'], p['eln1_b'], p['wes'], p['bes'])

  qs, kvd = _gather2(q, kv, src, dst)

  num3, uw3 = _edge_numuw(qs, kvd, ep)

  src2 = src.reshape(E // SZ, SZ)
  dens3, den = _den_kernel(num3, src2)

  attn, out_edges = _attn_ffn(num3, dens3, edge_states, p['weo'], p['beo'],
                              p['eln2_g'], p['eln2_b'],
                              p['ef1_w'], p['ef1_b'], p['ef2_w'], p['ef2_b'])

  uagg = _segsum_split(uw3, src2)

  out_nodes = _node_post(uagg, den, node_states, p['wno'], p['bno'],
                         p['nln2_g'], p['nln2_b'],
                         p['nf1_w'], p['nf1_b'], p['nf2_w'], p['nf2_b'])

  return (out_nodes, out_edges, attn)


# speedup vs baseline: 1.2490x; 1.2490x over previous
import math

import jax
import jax.numpy as jnp
from jax import lax
from jax.experimental import pallas as pl
from jax.experimental.pallas import tpu as pltpu
from jax.experimental.pallas import tpu_sc as plsc

N = 10000
E = 320000
D = 128
H = 8
DH = 16

NC = 2
NS = 16
NW = NC * NS
EPW = E // NW
EPT = E // NS
CH = 80
NCHUNK = EPW // CH
NCH2 = EPT // CH
GB = 128
NGB = EPW // GB
GREM = EPW - NGB * GB
NZC = 400
NZN = N // NZC
W2 = D // NC

_mesh = plsc.VectorSubcoreMesh(
    core_axis_name="c", subcore_axis_name="s", num_cores=NC, num_subcores=NS)
_sc_params = pltpu.CompilerParams(use_tc_tiling_on_sc=False)
_sc_params_tiled = pltpu.CompilerParams(use_tc_tiling_on_sc=True)


def _ln_block(x, g, b):
  ones = jnp.ones((D, 1), jnp.float32)
  s1 = jnp.dot(x, ones, preferred_element_type=jnp.float32)
  s2 = jnp.dot(x * x, ones, preferred_element_type=jnp.float32)
  m = s1 * (1.0 / D)
  var = s2 * (1.0 / D) - m * m
  return (x - m) * jax.lax.rsqrt(var + 1e-5) * g + b


def _head_sel():
  r = lax.broadcasted_iota(jnp.int32, (D, 16), 0) // DH
  c = lax.broadcasted_iota(jnp.int32, (D, 16), 1)
  return (r == c).astype(jnp.float32)


def _node_pre_body(x_ref, g_ref, b_ref, wq_ref, bq_ref, wk_ref, bk_ref,
                   wv_ref, bv_ref, q_ref, kv_ref):
  x = x_ref[...]
  xn = _ln_block(x, g_ref[...], b_ref[...])
  q = jnp.dot(xn, wq_ref[...], preferred_element_type=jnp.float32) + bq_ref[...]
  k = jnp.dot(xn, wk_ref[...], preferred_element_type=jnp.float32) + bk_ref[...]
  v = jnp.dot(xn, wv_ref[...], preferred_element_type=jnp.float32) + bv_ref[...]
  q_ref[...] = q
  kv_ref[...] = jnp.concatenate([k, v * xn], axis=1)


def _node_pre(x, g, b, wq, bq, wk, bk, wv, bv):
  blk = 1000
  grid = N // blk
  row = pl.BlockSpec((blk, D), lambda i: (i, 0))
  full = pl.BlockSpec((D, D), lambda i: (0, 0))
  vec = pl.BlockSpec((D,), lambda i: (0,))
  return pl.pallas_call(
      _node_pre_body,
      grid=(grid,),
      in_specs=[row, vec, vec, full, vec, full, vec, full, vec],
      out_specs=[row, pl.BlockSpec((blk, 2 * D), lambda i: (i, 0))],
      out_shape=[jax.ShapeDtypeStruct((N, D), jnp.float32),
                 jax.ShapeDtypeStruct((N, 2 * D), jnp.float32)],
  )(x, g, b, wq, bq, wk, bk, wv, bv)


def _edge_ep_body(e_ref, g_ref, b_ref, wes_ref, bes_ref, ep_ref):
  en = _ln_block(e_ref[...], g_ref[...], b_ref[...])
  esh = (jnp.dot(en, wes_ref[...], preferred_element_type=jnp.float32)
         + bes_ref[...])
  ep_ref[...] = jnp.dot(esh * en, _head_sel(),
                        preferred_element_type=jnp.float32)


def _edge_ep(e, g, b, wes, bes):
  blk = 8000
  grid = E // blk
  row = pl.BlockSpec((blk, D), lambda i: (i, 0))
  full = pl.BlockSpec((D, D), lambda i: (0, 0))
  vec = pl.BlockSpec((D,), lambda i: (0,))
  out = pl.BlockSpec((blk, 16), lambda i: (i, 0))
  return pl.pallas_call(
      _edge_ep_body,
      grid=(grid,),
      in_specs=[row, vec, vec, full, vec],
      out_specs=out,
      out_shape=jax.ShapeDtypeStruct((E, 16), jnp.float32),
  )(e, g, b, wes, bes)


def _gather2_body(q_hbm, kv_hbm, src_hbm, dst_hbm,
                  qs_out, kvd_out,
                  si, di, qb0, qb1, kvb0, kvb1, sg, so):
  wid = lax.axis_index("s") * NC + lax.axis_index("c")
  base0 = wid * EPW
  qbs = (qb0, qb1)
  kvbs = (kvb0, kvb1)

  def body(g, _):
    base = base0 + g * (2 * GB)

    @pl.when(g > 0)
    def _():
      for j in range(2):
        pltpu.make_async_copy(qbs[j], qs_out.at[pl.ds(base, GB)], so).wait()
        pltpu.make_async_copy(kvbs[j], kvd_out.at[pl.ds(base, GB)], so).wait()

    pltpu.sync_copy(src_hbm.at[pl.ds(base, 2 * GB)], si)
    pltpu.sync_copy(dst_hbm.at[pl.ds(base, 2 * GB)], di)
    copies = []
    for j in range(2):
      copies.append(
          pltpu.async_copy(q_hbm.at[si.at[pl.ds(j * GB, GB)]], qbs[j], sg))
      copies.append(
          pltpu.async_copy(kv_hbm.at[di.at[pl.ds(j * GB, GB)]], kvbs[j], sg))
    for cp in copies:
      cp.wait()
    for j in range(2):
      pltpu.async_copy(qbs[j], qs_out.at[pl.ds(base + j * GB, GB)], so)
      pltpu.async_copy(kvbs[j], kvd_out.at[pl.ds(base + j * GB, GB)], so)
    return 0

  lax.fori_loop(0, NGB // 2, body, 0)
  for j in range(2):
    pltpu.make_async_copy(qbs[j], qs_out.at[pl.ds(base0, GB)], so).wait()
    pltpu.make_async_copy(kvbs[j], kvd_out.at[pl.ds(base0, GB)], so).wait()

  rbase = base0 + NGB * GB
  pltpu.sync_copy(src_hbm.at[pl.ds(rbase, GREM)], si.at[pl.ds(0, GREM)])
  pltpu.sync_copy(dst_hbm.at[pl.ds(rbase, GREM)], di.at[pl.ds(0, GREM)])
  cq = pltpu.async_copy(q_hbm.at[si.at[pl.ds(0, GREM)]],
                        qb0.at[pl.ds(0, GREM)], sg)
  ckv = pltpu.async_copy(kv_hbm.at[di.at[pl.ds(0, GREM)]],
                         kvb0.at[pl.ds(0, GREM)], sg)
  cq.wait()
  ckv.wait()
  pltpu.sync_copy(qb0.at[pl.ds(0, GREM)], qs_out.at[pl.ds(rbase, GREM)])
  pltpu.sync_copy(kvb0.at[pl.ds(0, GREM)], kvd_out.at[pl.ds(rbase, GREM)])


def _gather2(q, kv, src, dst):
  f = pl.kernel(
      _gather2_body,
      out_type=[jax.ShapeDtypeStruct((E, D), jnp.float32),
                jax.ShapeDtypeStruct((E, 2 * D), jnp.float32)],
      mesh=_mesh,
      compiler_params=_sc_params_tiled,
      scratch_types=[
          pltpu.VMEM((2 * GB,), jnp.int32),
          pltpu.VMEM((2 * GB,), jnp.int32),
          pltpu.VMEM((GB, D), jnp.float32),
          pltpu.VMEM((GB, D), jnp.float32),
          pltpu.VMEM((GB, 2 * D), jnp.float32),
          pltpu.VMEM((GB, 2 * D), jnp.float32),
          pltpu.SemaphoreType.DMA,
          pltpu.SemaphoreType.DMA,
      ],
  )
  return f(q, kv, src, dst)


SZ = 100
EG = E // SZ


def _edge_numuw_body(qs_ref, kd_ref, vmd_ref, ep_ref, num_ref, uw_ref):
  blk = qs_ref.shape[0]
  sel = _head_sel()
  qk = jnp.dot(qs_ref[...] * kd_ref[...], sel,
               preferred_element_type=jnp.float32)
  num = jnp.exp((qk + ep_ref[...]) * (1.0 / math.sqrt(DH)))
  num_ref[...] = num.reshape(blk // SZ, SZ, 16)
  nb = jnp.dot(num, sel.T, preferred_element_type=jnp.float32)
  uw_ref[...] = (nb * vmd_ref[...]).reshape(blk // SZ, SZ, D)


def _edge_numuw(qs, kvd, ep):
  blk = 8000
  bg = blk // SZ
  grid = E // blk
  row = pl.BlockSpec((blk, D), lambda i: (i, 0))
  n16 = pl.BlockSpec((blk, 16), lambda i: (i, 0))
  kcol = pl.BlockSpec((blk, D), lambda i: (i, 0))
  vcol = pl.BlockSpec((blk, D), lambda i: (i, 1))
  return pl.pallas_call(
      _edge_numuw_body,
      grid=(grid,),
      in_specs=[row, kcol, vcol, n16],
      out_specs=[pl.BlockSpec((bg, SZ, 16), lambda i: (i, 0, 0)),
                 pl.BlockSpec((bg, SZ, D), lambda i: (i, 0, 0))],
      out_shape=[jax.ShapeDtypeStruct((EG, SZ, 16), jnp.float32),
                 jax.ShapeDtypeStruct((EG, SZ, D), jnp.float32)],
  )(qs, kvd, kvd, ep)


GQ = 10
NSC_T = EPT // SZ
NG_SCAT = NSC_T // GQ
NSC_W = EPW // SZ
NG_GATH = NSC_W // GQ


def _den_body(num3, src2, dens3_out, den_out,
              is0, is1, vs0, vs1, db0, db1, zb, acc, s0, s1, gs0, gs1):
  cid = lax.axis_index("c")
  sid = lax.axis_index("s")
  wid = sid * NC + cid
  isl = (is0, is1)
  vsl = (vs0, vs1)
  dbl = (db0, db1)
  ssem = (s0, s1)
  gsem = (gs0, gs1)

  zb[...] = jnp.zeros(zb.shape, jnp.float32)
  for j in range((NZN + NS - 1) // NS):
    ci = sid + j * NS
    @pl.when(ci < NZN)
    def _():
      pltpu.sync_copy(zb, acc.at[pl.ds(ci * NZC, NZC)])
  plsc.subcore_barrier()

  cbase = sid * NSC_T

  def abody(t, _):
    for j in range(2):
      gg = 2 * t + j
      cb = cbase + gg * GQ
      @pl.when(t > 0)
      def _():
        pltpu.make_async_copy(num3.at[pl.ds(cb, GQ)], vsl[j], ssem[j]).wait()
      pltpu.sync_copy(src2.at[pl.ds(cb, GQ)], isl[j])
      pltpu.sync_copy(num3.at[pl.ds(cb, GQ)], vsl[j])
      for jj in range(GQ):
        pltpu.async_copy(vsl[j].at[jj], acc.at[isl[j].at[jj]], ssem[j],
                         add=True)
    return 0

  lax.fori_loop(0, NG_SCAT // 2, abody, 0)
  for j in range(2):
    pltpu.make_async_copy(num3.at[pl.ds(cbase, GQ)], vsl[j], ssem[j]).wait()
  plsc.subcore_barrier()

  gbase = wid * NSC_W
  pltpu.sync_copy(src2.at[pl.ds(gbase, GQ)], is0)
  for jj in range(GQ):
    pltpu.async_copy(acc.at[is0.at[jj]], db0.at[jj], gs0)

  def gbody(t, _):
    for j in range(2):
      gg = 2 * t + j
      cb = gbase + gg * GQ
      @pl.when(gg + 1 < NG_GATH)
      def _():
        pltpu.sync_copy(src2.at[pl.ds(cb + GQ, GQ)], isl[1 - j])
        for jj in range(GQ):
          pltpu.async_copy(acc.at[isl[1 - j].at[jj]], dbl[1 - j].at[jj],
                           gsem[1 - j])
      pltpu.make_async_copy(num3.at[pl.ds(cb, GQ)], dbl[j], gsem[j]).wait()
      pltpu.sync_copy(dbl[j], dens3_out.at[pl.ds(cb, GQ)])
    return 0

  lax.fori_loop(0, NG_GATH // 2, gbody, 0)

  for j in range((NZN + NS - 1) // NS):
    ci = sid + j * NS
    @pl.when(jnp.logical_and(ci < NZN, (ci % NC) == cid))
    def _():
      pltpu.sync_copy(acc.at[pl.ds(ci * NZC, NZC)],
                      den_out.at[pl.ds(ci * NZC, NZC)])


def _den_kernel(num3, src2):
  f = pl.kernel(
      _den_body,
      out_type=[jax.ShapeDtypeStruct((E // SZ, SZ, 16), jnp.float32),
                jax.ShapeDtypeStruct((N, 16), jnp.float32)],
      mesh=_mesh,
      compiler_params=_sc_params,
      scratch_types=[
          pltpu.VMEM((GQ, SZ), jnp.int32),
          pltpu.VMEM((GQ, SZ), jnp.int32),
          pltpu.VMEM((GQ, SZ, 16), jnp.float32),
          pltpu.VMEM((GQ, SZ, 16), jnp.float32),
          pltpu.VMEM((GQ, SZ, 16), jnp.float32),
          pltpu.VMEM((GQ, SZ, 16), jnp.float32),
          pltpu.VMEM((NZC, 16), jnp.float32),
          pltpu.VMEM_SHARED((N, 16), jnp.float32),
          pltpu.SemaphoreType.DMA,
          pltpu.SemaphoreType.DMA,
          pltpu.SemaphoreType.DMA,
          pltpu.SemaphoreType.DMA,
      ],
  )
  return f(num3, src2)


GQA = 4
NG_AGG = NSC_T // GQA


def _segsum_split_body(vals3, src2, out_hbm, is0, is1, vs0, vs1, zb, acc,
                       s0, s1):
  cid = lax.axis_index("c")
  sid = lax.axis_index("s")
  c0 = cid * W2
  isl = (is0, is1)
  vsl = (vs0, vs1)
  ssem = (s0, s1)

  zb[...] = jnp.zeros(zb.shape, jnp.float32)
  for j in range((NZN + NS - 1) // NS):
    ci = sid + j * NS
    @pl.when(ci < NZN)
    def _():
      pltpu.sync_copy(zb, acc.at[pl.ds(ci * NZC, NZC)])
  plsc.subcore_barrier()

  cbase = sid * NSC_T

  def abody(t, _):
    for j in range(2):
      gg = 2 * t + j
      cb = cbase + gg * GQA
      @pl.when(t > 0)
      def _():
        pltpu.make_async_copy(
            vals3.at[pl.ds(cb, GQA), :, pl.ds(c0, W2)], vsl[j],
            ssem[j]).wait()
      pltpu.sync_copy(src2.at[pl.ds(cb, GQA)], isl[j])
      pltpu.sync_copy(vals3.at[pl.ds(cb, GQA), :, pl.ds(c0, W2)], vsl[j])
      for jj in range(GQA):
        pltpu.async_copy(vsl[j].at[jj], acc.at[isl[j].at[jj]], ssem[j],
                         add=True)
    return 0

  lax.fori_loop(0, NG_AGG // 2, abody, 0)
  for j in range(2):
    pltpu.make_async_copy(
        vals3.at[pl.ds(cbase, GQA), :, pl.ds(c0, W2)], vsl[j],
        ssem[j]).wait()
  plsc.subcore_barrier()
  for j in range((NZN + NS - 1) // NS):
    ci = sid + j * NS
    @pl.when(ci < NZN)
    def _():
      pltpu.sync_copy(acc.at[pl.ds(ci * NZC, NZC)],
                      out_hbm.at[pl.ds(ci * NZC, NZC), pl.ds(c0, W2)])


def _segsum_split(vals3, src2):
  f = pl.kernel(
      _segsum_split_body,
      out_type=jax.ShapeDtypeStruct((N, D), jnp.float32),
      mesh=_mesh,
      compiler_params=_sc_params,
      scratch_types=[
          pltpu.VMEM((GQA, SZ), jnp.int32),
          pltpu.VMEM((GQA, SZ), jnp.int32),
          pltpu.VMEM((GQA, SZ, W2), jnp.float32),
          pltpu.VMEM((GQA, SZ, W2), jnp.float32),
          pltpu.VMEM((NZC, W2), jnp.float32),
          pltpu.VMEM_SHARED((N, W2), jnp.float32),
          pltpu.SemaphoreType.DMA,
          pltpu.SemaphoreType.DMA,
      ],
  )
  return f(vals3, src2)


def _attn_ffn_body(num_ref, den_ref, e_ref, weo_ref, beo_ref,
                   g_ref, b_ref, w1_ref, b1_ref, w2_ref, b2_ref,
                   attn_ref, oe_ref):
  blk = e_ref.shape[0]
  num = num_ref[...].reshape(blk, 16)
  den = den_ref[...].reshape(blk, 16)
  attn16 = num / (den + 1e-12)
  attn = attn16[:, :H]
  attn_ref[...] = attn
  eau = (jnp.dot(attn, weo_ref[...],
                 preferred_element_type=jnp.float32) + beo_ref[...])
  es = e_ref[...] + eau
  x = _ln_block(es, g_ref[...], b_ref[...])
  h1 = jax.nn.relu(
      jnp.dot(x, w1_ref[...], preferred_element_type=jnp.float32) + b1_ref[...])
  ef = (jnp.dot(h1, w2_ref[...], preferred_element_type=jnp.float32)
        + b2_ref[...])
  oe_ref[...] = es + ef


def _attn_ffn(num, den_s, e, weo, beo, g, b, w1, b1, w2, b2):
  blk = 8000
  bg = blk // SZ
  grid = E // blk
  row = pl.BlockSpec((blk, D), lambda i: (i, 0))
  n16 = pl.BlockSpec((bg, SZ, 16), lambda i: (i, 0, 0))
  vec = pl.BlockSpec((D,), lambda i: (0,))
  return pl.pallas_call(
      _attn_ffn_body,
      grid=(grid,),
      in_specs=[n16, n16, row,
                pl.BlockSpec((H, D), lambda i: (0, 0)), vec,
                vec, vec,
                pl.BlockSpec((D, 2 * D), lambda i: (0, 0)),
                pl.BlockSpec((2 * D,), lambda i: (0,)),
                pl.BlockSpec((2 * D, D), lambda i: (0, 0)), vec],
      out_specs=[pl.BlockSpec((blk, H), lambda i: (i, 0)), row],
      out_shape=[jax.ShapeDtypeStruct((E, H), jnp.float32),
                 jax.ShapeDtypeStruct((E, D), jnp.float32)],
  )(num, den_s, e, weo, beo, g, b, w1, b1, w2, b2)


def _node_post_body(u_ref, den_ref, x_ref, wno_ref, bno_ref, g_ref, b_ref,
                    w1_ref, b1_ref, w2_ref, b2_ref, o_ref):
  den_b = jnp.dot(den_ref[...], _head_sel().T,
                  preferred_element_type=jnp.float32)
  agg = u_ref[...] / (den_b + 1e-30)
  nau = (jnp.dot(agg, wno_ref[...], preferred_element_type=jnp.float32)
         + bno_ref[...])
  ns = x_ref[...] + nau
  x = _ln_block(ns, g_ref[...], b_ref[...])
  h1 = jax.nn.relu(
      jnp.dot(x, w1_ref[...], preferred_element_type=jnp.float32) + b1_ref[...])
  nf = (jnp.dot(h1, w2_ref[...], preferred_element_type=jnp.float32)
        + b2_ref[...])
  o_ref[...] = ns + nf


def _node_post(u, den, x, wno, bno, g, b, w1, b1, w2, b2):
  blk = 1000
  grid = N // blk
  row = pl.BlockSpec((blk, D), lambda i: (i, 0))
  full = pl.BlockSpec((D, D), lambda i: (0, 0))
  vec = pl.BlockSpec((D,), lambda i: (0,))
  return pl.pallas_call(
      _node_post_body,
      grid=(grid,),
      in_specs=[row, pl.BlockSpec((blk, 16), lambda i: (i, 0)),
                row, full, vec, vec, vec,
                pl.BlockSpec((D, 2 * D), lambda i: (0, 0)),
                pl.BlockSpec((2 * D,), lambda i: (0,)),
                pl.BlockSpec((2 * D, D), lambda i: (0, 0)), vec],
      out_specs=row,
      out_shape=jax.ShapeDtypeStruct((N, D), jnp.float32),
  )(u, den, x, wno, bno, g, b, w1, b1, w2, b2)


@jax.jit
def kernel(node_states, edge_index, edge_states, params):
  p = params
  src = edge_index[0]
  dst = edge_index[1]

  q, kv = _node_pre(node_states, p['nln1_g'], p['nln1_b'],
                    p['wq'], p['bq'], p['wk'], p['bk'], p['wv'], p['bv'])

  ep = _edge_ep(edge_states, p['eln1_g'], p['eln1_b'], p['wes'], p['bes'])

  qs, kvd = _gather2(q, kv, src, dst)

  num3, uw3 = _edge_numuw(qs, kvd, ep)

  src2 = src.reshape(E // SZ, SZ)
  dens3, den = _den_kernel(num3, src2)

  attn, out_edges = _attn_ffn(num3, dens3, edge_states, p['weo'], p['beo'],
                              p['eln2_g'], p['eln2_b'],
                              p['ef1_w'], p['ef1_b'], p['ef2_w'], p['ef2_b'])

  uagg = _segsum_split(uw3, src2)

  out_nodes = _node_post(uagg, den, node_states, p['wno'], p['bno'],
                         p['nln2_g'], p['nln2_b'],
                         p['nf1_w'], p['nf1_b'], p['nf2_w'], p['nf2_b'])

  return (out_nodes, out_edges, attn)


# speedup vs baseline: 1.4890x; 1.1922x over previous
import math

import jax
import jax.numpy as jnp
from jax import lax
from jax.experimental import pallas as pl
from jax.experimental.pallas import tpu as pltpu
from jax.experimental.pallas import tpu_sc as plsc

N = 10000
E = 320000
D = 128
H = 8
DH = 16

NC = 2
NS = 16
NW = NC * NS
EPW = E // NW
EPT = E // NS
CH = 80
NCHUNK = EPW // CH
NCH2 = EPT // CH
GB = 128
NGB = EPW // GB
GREM = EPW - NGB * GB
NZC = 400
NZN = N // NZC
W2 = D // NC

_mesh = plsc.VectorSubcoreMesh(
    core_axis_name="c", subcore_axis_name="s", num_cores=NC, num_subcores=NS)
_sc_params = pltpu.CompilerParams(use_tc_tiling_on_sc=False)
_sc_params_tiled = pltpu.CompilerParams(use_tc_tiling_on_sc=True)


def _ln_block(x, g, b):
  ones = jnp.ones((D, 1), jnp.float32)
  s1 = jnp.dot(x, ones, preferred_element_type=jnp.float32)
  s2 = jnp.dot(x * x, ones, preferred_element_type=jnp.float32)
  m = s1 * (1.0 / D)
  var = s2 * (1.0 / D) - m * m
  return (x - m) * jax.lax.rsqrt(var + 1e-5) * g + b


def _head_sel(w=16):
  r = lax.broadcasted_iota(jnp.int32, (D, w), 0) // DH
  c = lax.broadcasted_iota(jnp.int32, (D, w), 1)
  return (r == c).astype(jnp.float32)


def _node_pre_body(x_ref, g_ref, b_ref, wq_ref, bq_ref, wk_ref, bk_ref,
                   wv_ref, bv_ref, q_ref, kv_ref):
  x = x_ref[...]
  xn = _ln_block(x, g_ref[...], b_ref[...])
  q = jnp.dot(xn, wq_ref[...], preferred_element_type=jnp.float32) + bq_ref[...]
  k = jnp.dot(xn, wk_ref[...], preferred_element_type=jnp.float32) + bk_ref[...]
  v = jnp.dot(xn, wv_ref[...], preferred_element_type=jnp.float32) + bv_ref[...]
  q_ref[...] = q
  kv_ref[...] = jnp.concatenate([k, v * xn], axis=1)


def _node_pre(x, g, b, wq, bq, wk, bk, wv, bv):
  blk = 1000
  grid = N // blk
  row = pl.BlockSpec((blk, D), lambda i: (i, 0))
  full = pl.BlockSpec((D, D), lambda i: (0, 0))
  vec = pl.BlockSpec((D,), lambda i: (0,))
  return pl.pallas_call(
      _node_pre_body,
      grid=(grid,),
      in_specs=[row, vec, vec, full, vec, full, vec, full, vec],
      out_specs=[row, pl.BlockSpec((blk, 2 * D), lambda i: (i, 0))],
      out_shape=[jax.ShapeDtypeStruct((N, D), jnp.float32),
                 jax.ShapeDtypeStruct((N, 2 * D), jnp.float32)],
  )(x, g, b, wq, bq, wk, bk, wv, bv)


def _edge_ep_body(e_ref, g_ref, b_ref, wes_ref, bes_ref, ep_ref):
  en = _ln_block(e_ref[...], g_ref[...], b_ref[...])
  esh = (jnp.dot(en, wes_ref[...], preferred_element_type=jnp.float32)
         + bes_ref[...])
  ep_ref[...] = jnp.dot(esh * en, _head_sel(D),
                        preferred_element_type=jnp.float32)


def _edge_ep(e, g, b, wes, bes):
  blk = 8000
  grid = E // blk
  row = pl.BlockSpec((blk, D), lambda i: (i, 0))
  full = pl.BlockSpec((D, D), lambda i: (0, 0))
  vec = pl.BlockSpec((D,), lambda i: (0,))
  return pl.pallas_call(
      _edge_ep_body,
      grid=(grid,),
      in_specs=[row, vec, vec, full, vec],
      out_specs=row,
      out_shape=jax.ShapeDtypeStruct((E, D), jnp.float32),
  )(e, g, b, wes, bes)


def _gather2_body(q_hbm, kv_hbm, src_hbm, dst_hbm,
                  qs_out, kvd_out,
                  si, di, qb0, qb1, kvb0, kvb1, sg, so):
  wid = lax.axis_index("s") * NC + lax.axis_index("c")
  base0 = wid * EPW
  qbs = (qb0, qb1)
  kvbs = (kvb0, kvb1)

  def body(g, _):
    base = base0 + g * (2 * GB)

    @pl.when(g > 0)
    def _():
      for j in range(2):
        pltpu.make_async_copy(qbs[j], qs_out.at[pl.ds(base, GB)], so).wait()
        pltpu.make_async_copy(kvbs[j], kvd_out.at[pl.ds(base, GB)], so).wait()

    pltpu.sync_copy(src_hbm.at[pl.ds(base, 2 * GB)], si)
    pltpu.sync_copy(dst_hbm.at[pl.ds(base, 2 * GB)], di)
    copies = []
    for j in range(2):
      copies.append(
          pltpu.async_copy(q_hbm.at[si.at[pl.ds(j * GB, GB)]], qbs[j], sg))
      copies.append(
          pltpu.async_copy(kv_hbm.at[di.at[pl.ds(j * GB, GB)]], kvbs[j], sg))
    for cp in copies:
      cp.wait()
    for j in range(2):
      pltpu.async_copy(qbs[j], qs_out.at[pl.ds(base + j * GB, GB)], so)
      pltpu.async_copy(kvbs[j], kvd_out.at[pl.ds(base + j * GB, GB)], so)
    return 0

  lax.fori_loop(0, NGB // 2, body, 0)
  for j in range(2):
    pltpu.make_async_copy(qbs[j], qs_out.at[pl.ds(base0, GB)], so).wait()
    pltpu.make_async_copy(kvbs[j], kvd_out.at[pl.ds(base0, GB)], so).wait()

  rbase = base0 + NGB * GB
  pltpu.sync_copy(src_hbm.at[pl.ds(rbase, GREM)], si.at[pl.ds(0, GREM)])
  pltpu.sync_copy(dst_hbm.at[pl.ds(rbase, GREM)], di.at[pl.ds(0, GREM)])
  cq = pltpu.async_copy(q_hbm.at[si.at[pl.ds(0, GREM)]],
                        qb0.at[pl.ds(0, GREM)], sg)
  ckv = pltpu.async_copy(kv_hbm.at[di.at[pl.ds(0, GREM)]],
                         kvb0.at[pl.ds(0, GREM)], sg)
  cq.wait()
  ckv.wait()
  pltpu.sync_copy(qb0.at[pl.ds(0, GREM)], qs_out.at[pl.ds(rbase, GREM)])
  pltpu.sync_copy(kvb0.at[pl.ds(0, GREM)], kvd_out.at[pl.ds(rbase, GREM)])


def _gather2(q, kv, src, dst):
  f = pl.kernel(
      _gather2_body,
      out_type=[jax.ShapeDtypeStruct((E, D), jnp.float32),
                jax.ShapeDtypeStruct((E, 2 * D), jnp.float32)],
      mesh=_mesh,
      compiler_params=_sc_params_tiled,
      scratch_types=[
          pltpu.VMEM((2 * GB,), jnp.int32),
          pltpu.VMEM((2 * GB,), jnp.int32),
          pltpu.VMEM((GB, D), jnp.float32),
          pltpu.VMEM((GB, D), jnp.float32),
          pltpu.VMEM((GB, 2 * D), jnp.float32),
          pltpu.VMEM((GB, 2 * D), jnp.float32),
          pltpu.SemaphoreType.DMA,
          pltpu.SemaphoreType.DMA,
      ],
  )
  return f(q, kv, src, dst)


SZ = 100
EG = E // SZ


def _edge_numuw_body(qs_ref, kd_ref, vmd_ref, ep_ref, num_ref, uw_ref):
  sel = _head_sel(D)
  qk = jnp.dot(qs_ref[...] * kd_ref[...], sel,
               preferred_element_type=jnp.float32)
  num = jnp.exp((qk + ep_ref[...]) * (1.0 / math.sqrt(DH)))
  num_ref[...] = num
  nb = jnp.dot(num, sel.T, preferred_element_type=jnp.float32)
  uw_ref[...] = nb * vmd_ref[...]


def _edge_numuw(qs, kvd, ep):
  blk = 8000
  grid = E // blk
  row = pl.BlockSpec((blk, D), lambda i: (i, 0))
  kcol = pl.BlockSpec((blk, D), lambda i: (i, 0))
  vcol = pl.BlockSpec((blk, D), lambda i: (i, 1))
  return pl.pallas_call(
      _edge_numuw_body,
      grid=(grid,),
      in_specs=[row, kcol, vcol, row],
      out_specs=[row, row],
      out_shape=[jax.ShapeDtypeStruct((E, D), jnp.float32),
                 jax.ShapeDtypeStruct((E, D), jnp.float32)],
  )(qs, kvd, kvd, ep)


GQ = 10
NSC_T = EPT // SZ
NG_SCAT = NSC_T // GQ
NSC_W = EPW // SZ
NG_GATH = NSC_W // GQ


def _den_body(num2, src2, dens_out, den_out,
              is0, is1, vs0, vs1, db0, db1, zb, acc, s0, s1, gs0, gs1):
  cid = lax.axis_index("c")
  sid = lax.axis_index("s")
  wid = sid * NC + cid
  isl = (is0, is1)
  vsl = (vs0, vs1)
  dbl = (db0, db1)
  ssem = (s0, s1)
  gsem = (gs0, gs1)

  zb[...] = jnp.zeros(zb.shape, jnp.float32)
  for j in range((NZN + NS - 1) // NS):
    ci = sid + j * NS
    @pl.when(ci < NZN)
    def _():
      pltpu.sync_copy(zb, acc.at[pl.ds(ci * NZC, NZC)])
  plsc.subcore_barrier()

  cbase = sid * NSC_T

  def abody(t, _):
    for j in range(2):
      gg = 2 * t + j
      cb = cbase + gg * GQ
      eb = cb * SZ
      @pl.when(t > 0)
      def _():
        pltpu.make_async_copy(
            num2.at[pl.ds(eb, GQ * SZ), pl.ds(0, 16)], vsl[j],
            ssem[j]).wait()
      pltpu.sync_copy(src2.at[pl.ds(cb, GQ)], isl[j])
      pltpu.sync_copy(num2.at[pl.ds(eb, GQ * SZ), pl.ds(0, 16)], vsl[j])
      for jj in range(GQ):
        pltpu.async_copy(vsl[j].at[pl.ds(jj * SZ, SZ)],
                         acc.at[isl[j].at[jj]], ssem[j], add=True)
    return 0

  lax.fori_loop(0, NG_SCAT // 2, abody, 0)
  for j in range(2):
    pltpu.make_async_copy(
        num2.at[pl.ds(cbase * SZ, GQ * SZ), pl.ds(0, 16)], vsl[j],
        ssem[j]).wait()
  plsc.subcore_barrier()

  gbase = wid * NSC_W
  pltpu.sync_copy(src2.at[pl.ds(gbase, GQ)], is0)
  for jj in range(GQ):
    pltpu.async_copy(acc.at[is0.at[jj]], db0.at[pl.ds(jj * SZ, SZ)], gs0)

  def gbody(t, _):
    for j in range(2):
      gg = 2 * t + j
      cb = gbase + gg * GQ
      @pl.when(gg + 1 < NG_GATH)
      def _():
        pltpu.sync_copy(src2.at[pl.ds(cb + GQ, GQ)], isl[1 - j])
        for jj in range(GQ):
          pltpu.async_copy(acc.at[isl[1 - j].at[jj]],
                           dbl[1 - j].at[pl.ds(jj * SZ, SZ)], gsem[1 - j])
      pltpu.make_async_copy(
          num2.at[pl.ds(cb * SZ, GQ * SZ), pl.ds(0, 16)], dbl[j],
          gsem[j]).wait()
      pltpu.sync_copy(dbl[j],
                      dens_out.at[pl.ds(cb * SZ, GQ * SZ), pl.ds(0, 16)])
    return 0

  lax.fori_loop(0, NG_GATH // 2, gbody, 0)

  for j in range((NZN + NS - 1) // NS):
    ci = sid + j * NS
    @pl.when(jnp.logical_and(ci < NZN, (ci % NC) == cid))
    def _():
      pltpu.sync_copy(acc.at[pl.ds(ci * NZC, NZC)],
                      den_out.at[pl.ds(ci * NZC, NZC), pl.ds(0, 16)])


def _den_kernel(num2, src2):
  f = pl.kernel(
      _den_body,
      out_type=[jax.ShapeDtypeStruct((E, D), jnp.float32),
                jax.ShapeDtypeStruct((N, D), jnp.float32)],
      mesh=_mesh,
      compiler_params=_sc_params,
      scratch_types=[
          pltpu.VMEM((GQ, SZ), jnp.int32),
          pltpu.VMEM((GQ, SZ), jnp.int32),
          pltpu.VMEM((GQ * SZ, 16), jnp.float32),
          pltpu.VMEM((GQ * SZ, 16), jnp.float32),
          pltpu.VMEM((GQ * SZ, 16), jnp.float32),
          pltpu.VMEM((GQ * SZ, 16), jnp.float32),
          pltpu.VMEM((NZC, 16), jnp.float32),
          pltpu.VMEM_SHARED((N, 16), jnp.float32),
          pltpu.SemaphoreType.DMA,
          pltpu.SemaphoreType.DMA,
          pltpu.SemaphoreType.DMA,
          pltpu.SemaphoreType.DMA,
      ],
  )
  return f(num2, src2)


GQA = 4
NG_AGG = NSC_T // GQA


def _segsum_split_body(vals2, src2, out_hbm, is0, is1, vs0, vs1, zb, acc,
                       s0, s1):
  cid = lax.axis_index("c")
  sid = lax.axis_index("s")
  c0 = cid * W2
  isl = (is0, is1)
  vsl = (vs0, vs1)
  ssem = (s0, s1)

  zb[...] = jnp.zeros(zb.shape, jnp.float32)
  for j in range((NZN + NS - 1) // NS):
    ci = sid + j * NS
    @pl.when(ci < NZN)
    def _():
      pltpu.sync_copy(zb, acc.at[pl.ds(ci * NZC, NZC)])
  plsc.subcore_barrier()

  cbase = sid * NSC_T

  def abody(t, _):
    for j in range(2):
      gg = 2 * t + j
      cb = cbase + gg * GQA
      eb = cb * SZ
      @pl.when(t > 0)
      def _():
        pltpu.make_async_copy(
            vals2.at[pl.ds(eb, GQA * SZ), pl.ds(c0, W2)], vsl[j],
            ssem[j]).wait()
      pltpu.sync_copy(src2.at[pl.ds(cb, GQA)], isl[j])
      pltpu.sync_copy(vals2.at[pl.ds(eb, GQA * SZ), pl.ds(c0, W2)], vsl[j])
      for jj in range(GQA):
        pltpu.async_copy(vsl[j].at[pl.ds(jj * SZ, SZ)],
                         acc.at[isl[j].at[jj]], ssem[j], add=True)
    return 0

  lax.fori_loop(0, NG_AGG // 2, abody, 0)
  for j in range(2):
    pltpu.make_async_copy(
        vals2.at[pl.ds(cbase * SZ, GQA * SZ), pl.ds(c0, W2)], vsl[j],
        ssem[j]).wait()
  plsc.subcore_barrier()
  for j in range((NZN + NS - 1) // NS):
    ci = sid + j * NS
    @pl.when(ci < NZN)
    def _():
      pltpu.sync_copy(acc.at[pl.ds(ci * NZC, NZC)],
                      out_hbm.at[pl.ds(ci * NZC, NZC), pl.ds(c0, W2)])


def _segsum_split(vals2, src2):
  f = pl.kernel(
      _segsum_split_body,
      out_type=jax.ShapeDtypeStruct((N, D), jnp.float32),
      mesh=_mesh,
      compiler_params=_sc_params,
      scratch_types=[
          pltpu.VMEM((GQA, SZ), jnp.int32),
          pltpu.VMEM((GQA, SZ), jnp.int32),
          pltpu.VMEM((GQA * SZ, W2), jnp.float32),
          pltpu.VMEM((GQA * SZ, W2), jnp.float32),
          pltpu.VMEM((NZC, W2), jnp.float32),
          pltpu.VMEM_SHARED((N, W2), jnp.float32),
          pltpu.SemaphoreType.DMA,
          pltpu.SemaphoreType.DMA,
      ],
  )
  return f(vals2, src2)


def _attn_ffn_body(num_ref, den_ref, e_ref, weo_ref, beo_ref,
                   g_ref, b_ref, w1_ref, b1_ref, w2_ref, b2_ref,
                   attn_ref, oe_ref):
  attn16 = num_ref[...][:, :16] / (den_ref[...][:, :16] + 1e-12)
  attn = attn16[:, :H]
  attn_ref[...] = attn
  eau = (jnp.dot(attn, weo_ref[...],
                 preferred_element_type=jnp.float32) + beo_ref[...])
  es = e_ref[...] + eau
  x = _ln_block(es, g_ref[...], b_ref[...])
  h1 = jax.nn.relu(
      jnp.dot(x, w1_ref[...], preferred_element_type=jnp.float32) + b1_ref[...])
  ef = (jnp.dot(h1, w2_ref[...], preferred_element_type=jnp.float32)
        + b2_ref[...])
  oe_ref[...] = es + ef


def _attn_ffn(num, den_s, e, weo, beo, g, b, w1, b1, w2, b2):
  blk = 8000
  grid = E // blk
  row = pl.BlockSpec((blk, D), lambda i: (i, 0))
  n16 = row
  vec = pl.BlockSpec((D,), lambda i: (0,))
  return pl.pallas_call(
      _attn_ffn_body,
      grid=(grid,),
      in_specs=[n16, n16, row,
                pl.BlockSpec((H, D), lambda i: (0, 0)), vec,
                vec, vec,
                pl.BlockSpec((D, 2 * D), lambda i: (0, 0)),
                pl.BlockSpec((2 * D,), lambda i: (0,)),
                pl.BlockSpec((2 * D, D), lambda i: (0, 0)), vec],
      out_specs=[pl.BlockSpec((blk, H), lambda i: (i, 0)), row],
      out_shape=[jax.ShapeDtypeStruct((E, H), jnp.float32),
                 jax.ShapeDtypeStruct((E, D), jnp.float32)],
  )(num, den_s, e, weo, beo, g, b, w1, b1, w2, b2)


def _node_post_body(u_ref, den_ref, x_ref, wno_ref, bno_ref, g_ref, b_ref,
                    w1_ref, b1_ref, w2_ref, b2_ref, o_ref):
  den_b = jnp.dot(den_ref[...][:, :16], _head_sel().T,
                  preferred_element_type=jnp.float32)
  agg = u_ref[...] / (den_b + 1e-30)
  nau = (jnp.dot(agg, wno_ref[...], preferred_element_type=jnp.float32)
         + bno_ref[...])
  ns = x_ref[...] + nau
  x = _ln_block(ns, g_ref[...], b_ref[...])
  h1 = jax.nn.relu(
      jnp.dot(x, w1_ref[...], preferred_element_type=jnp.float32) + b1_ref[...])
  nf = (jnp.dot(h1, w2_ref[...], preferred_element_type=jnp.float32)
        + b2_ref[...])
  o_ref[...] = ns + nf


def _node_post(u, den, x, wno, bno, g, b, w1, b1, w2, b2):
  blk = 1000
  grid = N // blk
  row = pl.BlockSpec((blk, D), lambda i: (i, 0))
  full = pl.BlockSpec((D, D), lambda i: (0, 0))
  vec = pl.BlockSpec((D,), lambda i: (0,))
  return pl.pallas_call(
      _node_post_body,
      grid=(grid,),
      in_specs=[row, row,
                row, full, vec, vec, vec,
                pl.BlockSpec((D, 2 * D), lambda i: (0, 0)),
                pl.BlockSpec((2 * D,), lambda i: (0,)),
                pl.BlockSpec((2 * D, D), lambda i: (0, 0)), vec],
      out_specs=row,
      out_shape=jax.ShapeDtypeStruct((N, D), jnp.float32),
  )(u, den, x, wno, bno, g, b, w1, b1, w2, b2)


@jax.jit
def kernel(node_states, edge_index, edge_states, params):
  p = params
  src = edge_index[0]
  dst = edge_index[1]

  q, kv = _node_pre(node_states, p['nln1_g'], p['nln1_b'],
                    p['wq'], p['bq'], p['wk'], p['bk'], p['wv'], p['bv'])

  ep = _edge_ep(edge_states, p['eln1_g'], p['eln1_b'], p['wes'], p['bes'])

  qs, kvd = _gather2(q, kv, src, dst)

  num3, uw3 = _edge_numuw(qs, kvd, ep)

  src2 = src.reshape(E // SZ, SZ)
  dens3, den = _den_kernel(num3, src2)

  attn, out_edges = _attn_ffn(num3, dens3, edge_states, p['weo'], p['beo'],
                              p['eln2_g'], p['eln2_b'],
                              p['ef1_w'], p['ef1_b'], p['ef2_w'], p['ef2_b'])

  uagg = _segsum_split(uw3, src2)

  out_nodes = _node_post(uagg, den, node_states, p['wno'], p['bno'],
                         p['nln2_g'], p['nln2_b'],
                         p['nf1_w'], p['nf1_b'], p['nf2_w'], p['nf2_b'])

  return (out_nodes, out_edges, attn)


# speedup vs baseline: 1.5476x; 1.0394x over previous
import math

import jax
import jax.numpy as jnp
from jax import lax
from jax.experimental import pallas as pl
from jax.experimental.pallas import tpu as pltpu
from jax.experimental.pallas import tpu_sc as plsc

N = 10000
E = 320000
D = 128
H = 8
DH = 16

NC = 2
NS = 16
NW = NC * NS
EPW = E // NW
EPT = E // NS
CH = 80
NCHUNK = EPW // CH
NCH2 = EPT // CH
GB = 128
NGB = EPW // GB
GREM = EPW - NGB * GB
NZC = 400
NZN = N // NZC
W2 = D // NC

_mesh = plsc.VectorSubcoreMesh(
    core_axis_name="c", subcore_axis_name="s", num_cores=NC, num_subcores=NS)
_sc_params = pltpu.CompilerParams(use_tc_tiling_on_sc=False)
_sc_params_tiled = pltpu.CompilerParams(use_tc_tiling_on_sc=True)


def _ln_block(x, g, b):
  ones = jnp.ones((D, 1), jnp.float32)
  s1 = jnp.dot(x, ones, preferred_element_type=jnp.float32)
  s2 = jnp.dot(x * x, ones, preferred_element_type=jnp.float32)
  m = s1 * (1.0 / D)
  var = s2 * (1.0 / D) - m * m
  return (x - m) * jax.lax.rsqrt(var + 1e-5) * g + b


def _head_sel(w=16):
  r = lax.broadcasted_iota(jnp.int32, (D, w), 0) // DH
  c = lax.broadcasted_iota(jnp.int32, (D, w), 1)
  return (r == c).astype(jnp.float32)


def _node_pre_body(x_ref, g_ref, b_ref, wq_ref, bq_ref, wk_ref, bk_ref,
                   wv_ref, bv_ref, q_ref, kv_ref):
  x = x_ref[...]
  xn = _ln_block(x, g_ref[...], b_ref[...])
  q = jnp.dot(xn, wq_ref[...], preferred_element_type=jnp.float32) + bq_ref[...]
  k = jnp.dot(xn, wk_ref[...], preferred_element_type=jnp.float32) + bk_ref[...]
  v = jnp.dot(xn, wv_ref[...], preferred_element_type=jnp.float32) + bv_ref[...]
  q_ref[...] = q
  kv_ref[...] = jnp.concatenate([k, v * xn], axis=1)


def _node_pre(x, g, b, wq, bq, wk, bk, wv, bv):
  blk = 1000
  grid = N // blk
  row = pl.BlockSpec((blk, D), lambda i: (i, 0))
  full = pl.BlockSpec((D, D), lambda i: (0, 0))
  vec = pl.BlockSpec((D,), lambda i: (0,))
  return pl.pallas_call(
      _node_pre_body,
      grid=(grid,),
      in_specs=[row, vec, vec, full, vec, full, vec, full, vec],
      out_specs=[row, pl.BlockSpec((blk, 2 * D), lambda i: (i, 0))],
      out_shape=[jax.ShapeDtypeStruct((N, D), jnp.float32),
                 jax.ShapeDtypeStruct((N, 2 * D), jnp.float32)],
  )(x, g, b, wq, bq, wk, bk, wv, bv)


def _edge_ep_body(e_ref, g_ref, b_ref, wes_ref, bes_ref, ep_ref):
  en = _ln_block(e_ref[...], g_ref[...], b_ref[...])
  esh = (jnp.dot(en, wes_ref[...], preferred_element_type=jnp.float32)
         + bes_ref[...])
  ep_ref[...] = jnp.dot(esh * en, _head_sel(D),
                        preferred_element_type=jnp.float32)


def _edge_ep(e, g, b, wes, bes):
  blk = 8000
  grid = E // blk
  row = pl.BlockSpec((blk, D), lambda i: (i, 0))
  full = pl.BlockSpec((D, D), lambda i: (0, 0))
  vec = pl.BlockSpec((D,), lambda i: (0,))
  return pl.pallas_call(
      _edge_ep_body,
      grid=(grid,),
      in_specs=[row, vec, vec, full, vec],
      out_specs=row,
      out_shape=jax.ShapeDtypeStruct((E, D), jnp.float32),
  )(e, g, b, wes, bes)


def _gather2_body(q_hbm, kv_hbm, src_hbm, dst_hbm,
                  qs_out, kvd_out,
                  si, di, qb0, qb1, kvb0, kvb1, sg, so):
  wid = lax.axis_index("s") * NC + lax.axis_index("c")
  base0 = wid * EPW
  qbs = (qb0, qb1)
  kvbs = (kvb0, kvb1)

  def body(g, _):
    base = base0 + g * (2 * GB)

    @pl.when(g > 0)
    def _():
      for j in range(2):
        pltpu.make_async_copy(qbs[j], qs_out.at[pl.ds(base, GB)], so).wait()
        pltpu.make_async_copy(kvbs[j], kvd_out.at[pl.ds(base, GB)], so).wait()

    pltpu.sync_copy(src_hbm.at[pl.ds(base, 2 * GB)], si)
    pltpu.sync_copy(dst_hbm.at[pl.ds(base, 2 * GB)], di)
    copies = []
    for j in range(2):
      copies.append(
          pltpu.async_copy(q_hbm.at[si.at[pl.ds(j * GB, GB)]], qbs[j], sg))
      copies.append(
          pltpu.async_copy(kv_hbm.at[di.at[pl.ds(j * GB, GB)]], kvbs[j], sg))
    for cp in copies:
      cp.wait()
    for j in range(2):
      pltpu.async_copy(qbs[j], qs_out.at[pl.ds(base + j * GB, GB)], so)
      pltpu.async_copy(kvbs[j], kvd_out.at[pl.ds(base + j * GB, GB)], so)
    return 0

  lax.fori_loop(0, NGB // 2, body, 0)
  for j in range(2):
    pltpu.make_async_copy(qbs[j], qs_out.at[pl.ds(base0, GB)], so).wait()
    pltpu.make_async_copy(kvbs[j], kvd_out.at[pl.ds(base0, GB)], so).wait()

  rbase = base0 + NGB * GB
  pltpu.sync_copy(src_hbm.at[pl.ds(rbase, GREM)], si.at[pl.ds(0, GREM)])
  pltpu.sync_copy(dst_hbm.at[pl.ds(rbase, GREM)], di.at[pl.ds(0, GREM)])
  cq = pltpu.async_copy(q_hbm.at[si.at[pl.ds(0, GREM)]],
                        qb0.at[pl.ds(0, GREM)], sg)
  ckv = pltpu.async_copy(kv_hbm.at[di.at[pl.ds(0, GREM)]],
                         kvb0.at[pl.ds(0, GREM)], sg)
  cq.wait()
  ckv.wait()
  pltpu.sync_copy(qb0.at[pl.ds(0, GREM)], qs_out.at[pl.ds(rbase, GREM)])
  pltpu.sync_copy(kvb0.at[pl.ds(0, GREM)], kvd_out.at[pl.ds(rbase, GREM)])


def _gather2(q, kv, src, dst):
  f = pl.kernel(
      _gather2_body,
      out_type=[jax.ShapeDtypeStruct((E, D), jnp.float32),
                jax.ShapeDtypeStruct((E, 2 * D), jnp.float32)],
      mesh=_mesh,
      compiler_params=_sc_params_tiled,
      scratch_types=[
          pltpu.VMEM((2 * GB,), jnp.int32),
          pltpu.VMEM((2 * GB,), jnp.int32),
          pltpu.VMEM((GB, D), jnp.float32),
          pltpu.VMEM((GB, D), jnp.float32),
          pltpu.VMEM((GB, 2 * D), jnp.float32),
          pltpu.VMEM((GB, 2 * D), jnp.float32),
          pltpu.SemaphoreType.DMA,
          pltpu.SemaphoreType.DMA,
      ],
  )
  return f(q, kv, src, dst)


SZ = 100
EG = E // SZ


def _edge_numuw_body(qs_ref, kd_ref, vmd_ref, ep_ref, num_ref, uw_ref):
  sel = _head_sel(D)
  qk = jnp.dot(qs_ref[...] * kd_ref[...], sel,
               preferred_element_type=jnp.float32)
  num = jnp.exp((qk + ep_ref[...]) * (1.0 / math.sqrt(DH)))
  num_ref[...] = num
  nb = jnp.dot(num, sel.T, preferred_element_type=jnp.float32)
  uw_ref[...] = nb * vmd_ref[...]


def _edge_numuw(qs, kvd, ep):
  blk = 8000
  grid = E // blk
  row = pl.BlockSpec((blk, D), lambda i: (i, 0))
  kcol = pl.BlockSpec((blk, D), lambda i: (i, 0))
  vcol = pl.BlockSpec((blk, D), lambda i: (i, 1))
  return pl.pallas_call(
      _edge_numuw_body,
      grid=(grid,),
      in_specs=[row, kcol, vcol, row],
      out_specs=[row, row],
      out_shape=[jax.ShapeDtypeStruct((E, D), jnp.float32),
                 jax.ShapeDtypeStruct((E, D), jnp.float32)],
  )(qs, kvd, kvd, ep)


GQ = 10
NSC_T = EPT // SZ
NG_SCAT = NSC_T // GQ
NSC_W = EPW // SZ
NG_GATH = NSC_W // GQ


def _den_body(num2, src2, dens_out, den_out,
              is0, is1, vs0, vs1, db0, db1, zb, acc, s0, s1, gs0, gs1):
  cid = lax.axis_index("c")
  sid = lax.axis_index("s")
  wid = sid * NC + cid
  isl = (is0, is1)
  vsl = (vs0, vs1)
  dbl = (db0, db1)
  ssem = (s0, s1)
  gsem = (gs0, gs1)

  zb[...] = jnp.zeros(zb.shape, jnp.float32)
  for j in range((NZN + NS - 1) // NS):
    ci = sid + j * NS
    @pl.when(ci < NZN)
    def _():
      pltpu.sync_copy(zb, acc.at[pl.ds(ci * NZC, NZC)])
  plsc.subcore_barrier()

  cbase = sid * NSC_T

  def abody(t, _):
    for j in range(2):
      gg = 2 * t + j
      cb = cbase + gg * GQ
      eb = cb * SZ
      @pl.when(t > 0)
      def _():
        pltpu.make_async_copy(
            num2.at[pl.ds(eb, GQ * SZ), pl.ds(0, 16)], vsl[j],
            ssem[j]).wait()
      pltpu.sync_copy(src2.at[pl.ds(cb, GQ)], isl[j])
      pltpu.sync_copy(num2.at[pl.ds(eb, GQ * SZ), pl.ds(0, 16)], vsl[j])
      for jj in range(GQ):
        pltpu.async_copy(vsl[j].at[pl.ds(jj * SZ, SZ)],
                         acc.at[isl[j].at[jj]], ssem[j], add=True)
    return 0

  lax.fori_loop(0, NG_SCAT // 2, abody, 0)
  for j in range(2):
    pltpu.make_async_copy(
        num2.at[pl.ds(cbase * SZ, GQ * SZ), pl.ds(0, 16)], vsl[j],
        ssem[j]).wait()
  plsc.subcore_barrier()

  gbase = wid * NSC_W
  pltpu.sync_copy(src2.at[pl.ds(gbase, GQ)], is0)
  for jj in range(GQ):
    pltpu.async_copy(acc.at[is0.at[jj]], db0.at[pl.ds(jj * SZ, SZ)], gs0)

  def gbody(t, _):
    for j in range(2):
      gg = 2 * t + j
      cb = gbase + gg * GQ
      @pl.when(gg + 1 < NG_GATH)
      def _():
        pltpu.sync_copy(src2.at[pl.ds(cb + GQ, GQ)], isl[1 - j])
        for jj in range(GQ):
          pltpu.async_copy(acc.at[isl[1 - j].at[jj]],
                           dbl[1 - j].at[pl.ds(jj * SZ, SZ)], gsem[1 - j])
      pltpu.make_async_copy(
          num2.at[pl.ds(cb * SZ, GQ * SZ), pl.ds(0, 16)], dbl[j],
          gsem[j]).wait()
      pltpu.sync_copy(dbl[j],
                      dens_out.at[pl.ds(cb * SZ, GQ * SZ), pl.ds(0, 16)])
    return 0

  lax.fori_loop(0, NG_GATH // 2, gbody, 0)

  for j in range((NZN + NS - 1) // NS):
    ci = sid + j * NS
    @pl.when(jnp.logical_and(ci < NZN, (ci % NC) == cid))
    def _():
      pltpu.sync_copy(acc.at[pl.ds(ci * NZC, NZC)],
                      den_out.at[pl.ds(ci * NZC, NZC), pl.ds(0, 16)])


def _den_kernel(num2, src2):
  f = pl.kernel(
      _den_body,
      out_type=[jax.ShapeDtypeStruct((E, D), jnp.float32),
                jax.ShapeDtypeStruct((N, D), jnp.float32)],
      mesh=_mesh,
      compiler_params=_sc_params,
      scratch_types=[
          pltpu.VMEM((GQ, SZ), jnp.int32),
          pltpu.VMEM((GQ, SZ), jnp.int32),
          pltpu.VMEM((GQ * SZ, 16), jnp.float32),
          pltpu.VMEM((GQ * SZ, 16), jnp.float32),
          pltpu.VMEM((GQ * SZ, 16), jnp.float32),
          pltpu.VMEM((GQ * SZ, 16), jnp.float32),
          pltpu.VMEM((NZC, 16), jnp.float32),
          pltpu.VMEM_SHARED((N, 16), jnp.float32),
          pltpu.SemaphoreType.DMA,
          pltpu.SemaphoreType.DMA,
          pltpu.SemaphoreType.DMA,
          pltpu.SemaphoreType.DMA,
      ],
  )
  return f(num2, src2)


GQA = 4
NG_AGG = NSC_T // GQA


def _segsum_split_body(vals2, src2, out_hbm, is0, is1, vs0, vs1, zb, acc,
                       s0, s1):
  cid = lax.axis_index("c")
  sid = lax.axis_index("s")
  c0 = cid * W2
  isl = (is0, is1)
  vsl = (vs0, vs1)
  ssem = (s0, s1)

  zb[...] = jnp.zeros(zb.shape, jnp.float32)
  for j in range((NZN + NS - 1) // NS):
    ci = sid + j * NS
    @pl.when(ci < NZN)
    def _():
      pltpu.sync_copy(zb, acc.at[pl.ds(ci * NZC, NZC)])
  plsc.subcore_barrier()

  cbase = sid * NSC_T

  def abody(t, _):
    for j in range(2):
      gg = 2 * t + j
      cb = cbase + gg * GQA
      eb = cb * SZ
      @pl.when(t > 0)
      def _():
        pltpu.make_async_copy(
            vals2.at[pl.ds(eb, GQA * SZ), pl.ds(c0, W2)], vsl[j],
            ssem[j]).wait()
      pltpu.sync_copy(src2.at[pl.ds(cb, GQA)], isl[j])
      pltpu.sync_copy(vals2.at[pl.ds(eb, GQA * SZ), pl.ds(c0, W2)], vsl[j])
      for jj in range(GQA):
        pltpu.async_copy(vsl[j].at[pl.ds(jj * SZ, SZ)],
                         acc.at[isl[j].at[jj]], ssem[j], add=True)
    return 0

  lax.fori_loop(0, NG_AGG // 2, abody, 0)
  for j in range(2):
    pltpu.make_async_copy(
        vals2.at[pl.ds(cbase * SZ, GQA * SZ), pl.ds(c0, W2)], vsl[j],
        ssem[j]).wait()
  plsc.subcore_barrier()
  for j in range((NZN + NS - 1) // NS):
    ci = sid + j * NS
    @pl.when(ci < NZN)
    def _():
      pltpu.sync_copy(acc.at[pl.ds(ci * NZC, NZC)],
                      out_hbm.at[pl.ds(ci * NZC, NZC), pl.ds(c0, W2)])


def _segsum_split(vals2, src2):
  f = pl.kernel(
      _segsum_split_body,
      out_type=jax.ShapeDtypeStruct((N, D), jnp.float32),
      mesh=_mesh,
      compiler_params=_sc_params,
      scratch_types=[
          pltpu.VMEM((GQA, SZ), jnp.int32),
          pltpu.VMEM((GQA, SZ), jnp.int32),
          pltpu.VMEM((GQA * SZ, W2), jnp.float32),
          pltpu.VMEM((GQA * SZ, W2), jnp.float32),
          pltpu.VMEM((NZC, W2), jnp.float32),
          pltpu.VMEM_SHARED((N, W2), jnp.float32),
          pltpu.SemaphoreType.DMA,
          pltpu.SemaphoreType.DMA,
      ],
  )
  return f(vals2, src2)


def _attn_ffn_body(num_ref, den_ref, e_ref, weo_ref, beo_ref,
                   g_ref, b_ref, w1_ref, b1_ref, w2_ref, b2_ref,
                   attn_ref, oe_ref):
  attn128 = num_ref[...] / (den_ref[...] + 1e-12)
  attn_ref[...] = attn128
  eau = (jnp.dot(attn128, weo_ref[...],
                 preferred_element_type=jnp.float32) + beo_ref[...])
  es = e_ref[...] + eau
  x = _ln_block(es, g_ref[...], b_ref[...])
  h1 = jax.nn.relu(
      jnp.dot(x, w1_ref[...], preferred_element_type=jnp.float32) + b1_ref[...])
  ef = (jnp.dot(h1, w2_ref[...], preferred_element_type=jnp.float32)
        + b2_ref[...])
  oe_ref[...] = es + ef


def _attn_ffn(num, den_s, e, weo, beo, g, b, w1, b1, w2, b2):
  blk = 8000
  grid = E // blk
  row = pl.BlockSpec((blk, D), lambda i: (i, 0))
  n16 = row
  vec = pl.BlockSpec((D,), lambda i: (0,))
  return pl.pallas_call(
      _attn_ffn_body,
      grid=(grid,),
      in_specs=[n16, n16, row,
                pl.BlockSpec((D, D), lambda i: (0, 0)), vec,
                vec, vec,
                pl.BlockSpec((D, 2 * D), lambda i: (0, 0)),
                pl.BlockSpec((2 * D,), lambda i: (0,)),
                pl.BlockSpec((2 * D, D), lambda i: (0, 0)), vec],
      out_specs=[row, row],
      out_shape=[jax.ShapeDtypeStruct((E, D), jnp.float32),
                 jax.ShapeDtypeStruct((E, D), jnp.float32)],
  )(num, den_s, e, weo, beo, g, b, w1, b1, w2, b2)


def _node_post_body(u_ref, den_ref, x_ref, wno_ref, bno_ref, g_ref, b_ref,
                    w1_ref, b1_ref, w2_ref, b2_ref, o_ref):
  den_b = jnp.dot(den_ref[...][:, :16], _head_sel().T,
                  preferred_element_type=jnp.float32)
  agg = u_ref[...] / (den_b + 1e-30)
  nau = (jnp.dot(agg, wno_ref[...], preferred_element_type=jnp.float32)
         + bno_ref[...])
  ns = x_ref[...] + nau
  x = _ln_block(ns, g_ref[...], b_ref[...])
  h1 = jax.nn.relu(
      jnp.dot(x, w1_ref[...], preferred_element_type=jnp.float32) + b1_ref[...])
  nf = (jnp.dot(h1, w2_ref[...], preferred_element_type=jnp.float32)
        + b2_ref[...])
  o_ref[...] = ns + nf


def _node_post(u, den, x, wno, bno, g, b, w1, b1, w2, b2):
  blk = 1000
  grid = N // blk
  row = pl.BlockSpec((blk, D), lambda i: (i, 0))
  full = pl.BlockSpec((D, D), lambda i: (0, 0))
  vec = pl.BlockSpec((D,), lambda i: (0,))
  return pl.pallas_call(
      _node_post_body,
      grid=(grid,),
      in_specs=[row, row,
                row, full, vec, vec, vec,
                pl.BlockSpec((D, 2 * D), lambda i: (0, 0)),
                pl.BlockSpec((2 * D,), lambda i: (0,)),
                pl.BlockSpec((2 * D, D), lambda i: (0, 0)), vec],
      out_specs=row,
      out_shape=jax.ShapeDtypeStruct((N, D), jnp.float32),
  )(u, den, x, wno, bno, g, b, w1, b1, w2, b2)


@jax.jit
def kernel(node_states, edge_index, edge_states, params):
  p = params
  src = edge_index[0]
  dst = edge_index[1]

  q, kv = _node_pre(node_states, p['nln1_g'], p['nln1_b'],
                    p['wq'], p['bq'], p['wk'], p['bk'], p['wv'], p['bv'])

  ep = _edge_ep(edge_states, p['eln1_g'], p['eln1_b'], p['wes'], p['bes'])

  qs, kvd = _gather2(q, kv, src, dst)

  num3, uw3 = _edge_numuw(qs, kvd, ep)

  src2 = src.reshape(E // SZ, SZ)
  dens3, den = _den_kernel(num3, src2)

  weo_pad = jnp.pad(p['weo'], ((0, D - H), (0, 0)))
  attn128, out_edges = _attn_ffn(num3, dens3, edge_states, weo_pad, p['beo'],
                                 p['eln2_g'], p['eln2_b'],
                                 p['ef1_w'], p['ef1_b'],
                                 p['ef2_w'], p['ef2_b'])
  attn = attn128[:, :H]

  uagg = _segsum_split(uw3, src2)

  out_nodes = _node_post(uagg, den, node_states, p['wno'], p['bno'],
                         p['nln2_g'], p['nln2_b'],
                         p['nf1_w'], p['nf1_b'], p['nf2_w'], p['nf2_b'])

  return (out_nodes, out_edges, attn)


# speedup vs baseline: 1.6118x; 1.0415x over previous
import math

import jax
import jax.numpy as jnp
from jax import lax
from jax.experimental import pallas as pl
from jax.experimental.pallas import tpu as pltpu
from jax.experimental.pallas import tpu_sc as plsc

N = 10000
E = 320000
D = 128
H = 8
DH = 16

NC = 2
NS = 16
NW = NC * NS
EPW = E // NW
EPT = E // NS
CH = 80
NCHUNK = EPW // CH
NCH2 = EPT // CH
GB = 128
NGB = EPW // GB
GREM = EPW - NGB * GB
NZC = 400
NZN = N // NZC
W2 = D // NC

_mesh = plsc.VectorSubcoreMesh(
    core_axis_name="c", subcore_axis_name="s", num_cores=NC, num_subcores=NS)
_sc_params = pltpu.CompilerParams(use_tc_tiling_on_sc=False)
_sc_params_tiled = pltpu.CompilerParams(use_tc_tiling_on_sc=True)


def _ln_block(x, g, b):
  ones = jnp.ones((D, 1), jnp.float32)
  s1 = jnp.dot(x, ones, preferred_element_type=jnp.float32)
  s2 = jnp.dot(x * x, ones, preferred_element_type=jnp.float32)
  m = s1 * (1.0 / D)
  var = s2 * (1.0 / D) - m * m
  return (x - m) * jax.lax.rsqrt(var + 1e-5) * g + b


def _head_sel(w=16):
  r = lax.broadcasted_iota(jnp.int32, (D, w), 0) // DH
  c = lax.broadcasted_iota(jnp.int32, (D, w), 1)
  return (r == c).astype(jnp.float32)


def _node_pre_body(x_ref, g_ref, b_ref, wq_ref, bq_ref, wk_ref, bk_ref,
                   wv_ref, bv_ref, q_ref, kv_ref):
  x = x_ref[...]
  xn = _ln_block(x, g_ref[...], b_ref[...])
  q = jnp.dot(xn, wq_ref[...], preferred_element_type=jnp.float32) + bq_ref[...]
  k = jnp.dot(xn, wk_ref[...], preferred_element_type=jnp.float32) + bk_ref[...]
  v = jnp.dot(xn, wv_ref[...], preferred_element_type=jnp.float32) + bv_ref[...]
  q_ref[...] = q
  kv_ref[...] = jnp.concatenate([k, v * xn], axis=1)


def _node_pre(x, g, b, wq, bq, wk, bk, wv, bv):
  blk = 1000
  grid = N // blk
  row = pl.BlockSpec((blk, D), lambda i: (i, 0))
  full = pl.BlockSpec((D, D), lambda i: (0, 0))
  vec = pl.BlockSpec((D,), lambda i: (0,))
  return pl.pallas_call(
      _node_pre_body,
      grid=(grid,),
      in_specs=[row, vec, vec, full, vec, full, vec, full, vec],
      out_specs=[row, pl.BlockSpec((blk, 2 * D), lambda i: (i, 0))],
      out_shape=[jax.ShapeDtypeStruct((N, D), jnp.float32),
                 jax.ShapeDtypeStruct((N, 2 * D), jnp.float32)],
  )(x, g, b, wq, bq, wk, bk, wv, bv)


def _edge_ep_body(e_ref, g_ref, b_ref, wes_ref, bes_ref, ep_ref):
  en = _ln_block(e_ref[...], g_ref[...], b_ref[...])
  esh = (jnp.dot(en, wes_ref[...], preferred_element_type=jnp.float32)
         + bes_ref[...])
  ep_ref[...] = jnp.dot(esh * en, _head_sel(D),
                        preferred_element_type=jnp.float32)


def _edge_ep(e, g, b, wes, bes):
  blk = 8000
  grid = E // blk
  row = pl.BlockSpec((blk, D), lambda i: (i, 0))
  full = pl.BlockSpec((D, D), lambda i: (0, 0))
  vec = pl.BlockSpec((D,), lambda i: (0,))
  return pl.pallas_call(
      _edge_ep_body,
      grid=(grid,),
      in_specs=[row, vec, vec, full, vec],
      out_specs=row,
      out_shape=jax.ShapeDtypeStruct((E, D), jnp.float32),
  )(e, g, b, wes, bes)


GB = 80
NGR = EPW // GB


def _gather2_body(q_hbm, kv_hbm, src_hbm, dst_hbm,
                  qs_out, kvd_out,
                  si0, si1, si2, di0, di1, di2,
                  qb0, qb1, qb2, kvb0, kvb1, kvb2,
                  g0, g1, g2, o0, o1, o2):
  wid = lax.axis_index("s") * NC + lax.axis_index("c")
  base0 = wid * EPW
  sis = (si0, si1, si2)
  dis = (di0, di1, di2)
  qbs = (qb0, qb1, qb2)
  kvbs = (kvb0, kvb1, kvb2)
  gsem = (g0, g1, g2)
  osem = (o0, o1, o2)

  def step(i, j, first, has_prev):
    prev = (j - 1) % 3
    base = base0 + i * GB

    def drain_store(sl):
      pltpu.make_async_copy(qbs[sl], qs_out.at[pl.ds(base0, GB)],
                            osem[sl]).wait()
      pltpu.make_async_copy(kvbs[sl], kvd_out.at[pl.ds(base0, GB)],
                            osem[sl]).wait()

    def fire_prev_store():
      pltpu.make_async_copy(qbs[prev], qs_out.at[pl.ds(base - GB, GB)],
                            gsem[prev]).wait()
      pltpu.make_async_copy(kvbs[prev], kvd_out.at[pl.ds(base - GB, GB)],
                            gsem[prev]).wait()
      pltpu.async_copy(qbs[prev], qs_out.at[pl.ds(base - GB, GB)], osem[prev])
      pltpu.async_copy(kvbs[prev], kvd_out.at[pl.ds(base - GB, GB)],
                       osem[prev])

    if first:
      @pl.when(i >= 3)
      def _():
        drain_store(j)
    else:
      drain_store(j)
    pltpu.sync_copy(src_hbm.at[pl.ds(base, GB)], sis[j])
    pltpu.sync_copy(dst_hbm.at[pl.ds(base, GB)], dis[j])
    pltpu.async_copy(q_hbm.at[sis[j]], qbs[j], gsem[j])
    pltpu.async_copy(kv_hbm.at[dis[j]], kvbs[j], gsem[j])
    if has_prev:
      fire_prev_store()
    else:
      @pl.when(i >= 1)
      def _():
        fire_prev_store()

  def body(t, _):
    for j in range(3):
      i = 3 * t + j
      step(i, j, first=True, has_prev=(j != 0))
    return 0

  lax.fori_loop(0, (NGR - 2) // 3, body, 0)
  for i in (NGR - 2, NGR - 1):
    step(i, i % 3, first=False, has_prev=True)
  last = (NGR - 1) % 3
  lbase = base0 + (NGR - 1) * GB
  pltpu.make_async_copy(qbs[last], qs_out.at[pl.ds(lbase, GB)],
                        gsem[last]).wait()
  pltpu.make_async_copy(kvbs[last], kvd_out.at[pl.ds(lbase, GB)],
                        gsem[last]).wait()
  pltpu.sync_copy(qbs[last], qs_out.at[pl.ds(lbase, GB)])
  pltpu.sync_copy(kvbs[last], kvd_out.at[pl.ds(lbase, GB)])
  for sl in range(3):
    if sl != last:
      pltpu.make_async_copy(qbs[sl], qs_out.at[pl.ds(base0, GB)],
                            osem[sl]).wait()
      pltpu.make_async_copy(kvbs[sl], kvd_out.at[pl.ds(base0, GB)],
                            osem[sl]).wait()


def _gather2(q, kv, src, dst):
  f = pl.kernel(
      _gather2_body,
      out_type=[jax.ShapeDtypeStruct((E, D), jnp.float32),
                jax.ShapeDtypeStruct((E, 2 * D), jnp.float32)],
      mesh=_mesh,
      compiler_params=_sc_params_tiled,
      scratch_types=(
          [pltpu.VMEM((GB,), jnp.int32)] * 6
          + [pltpu.VMEM((GB, D), jnp.float32)] * 3
          + [pltpu.VMEM((GB, 2 * D), jnp.float32)] * 3
          + [pltpu.SemaphoreType.DMA] * 6
      ),
  )
  return f(q, kv, src, dst)


SZ = 100
EG = E // SZ


def _edge_numuw_body(qs_ref, kd_ref, vmd_ref, ep_ref, num_ref, uw_ref):
  sel = _head_sel(D)
  qk = jnp.dot(qs_ref[...] * kd_ref[...], sel,
               preferred_element_type=jnp.float32)
  num = jnp.exp((qk + ep_ref[...]) * (1.0 / math.sqrt(DH)))
  num_ref[...] = num
  nb = jnp.dot(num, sel.T, preferred_element_type=jnp.float32)
  uw_ref[...] = nb * vmd_ref[...]


def _edge_numuw(qs, kvd, ep):
  blk = 8000
  grid = E // blk
  row = pl.BlockSpec((blk, D), lambda i: (i, 0))
  kcol = pl.BlockSpec((blk, D), lambda i: (i, 0))
  vcol = pl.BlockSpec((blk, D), lambda i: (i, 1))
  return pl.pallas_call(
      _edge_numuw_body,
      grid=(grid,),
      in_specs=[row, kcol, vcol, row],
      out_specs=[row, row],
      out_shape=[jax.ShapeDtypeStruct((E, D), jnp.float32),
                 jax.ShapeDtypeStruct((E, D), jnp.float32)],
  )(qs, kvd, kvd, ep)


GQ = 10
NSC_T = EPT // SZ
NG_SCAT = NSC_T // GQ
NSC_W = EPW // SZ
NG_GATH = NSC_W // GQ


def _den_body(num2, src2, dens_out, den_out,
              is0, is1, vs0, vs1, db0, db1, zb, acc, s0, s1, gs0, gs1):
  cid = lax.axis_index("c")
  sid = lax.axis_index("s")
  wid = sid * NC + cid
  isl = (is0, is1)
  vsl = (vs0, vs1)
  dbl = (db0, db1)
  ssem = (s0, s1)
  gsem = (gs0, gs1)

  zb[...] = jnp.zeros(zb.shape, jnp.float32)
  for j in range((NZN + NS - 1) // NS):
    ci = sid + j * NS
    @pl.when(ci < NZN)
    def _():
      pltpu.sync_copy(zb, acc.at[pl.ds(ci * NZC, NZC)])
  plsc.subcore_barrier()

  cbase = sid * NSC_T

  def abody(t, _):
    for j in range(2):
      gg = 2 * t + j
      cb = cbase + gg * GQ
      eb = cb * SZ
      @pl.when(t > 0)
      def _():
        pltpu.make_async_copy(
            num2.at[pl.ds(eb, GQ * SZ), pl.ds(0, 16)], vsl[j],
            ssem[j]).wait()
      pltpu.sync_copy(src2.at[pl.ds(cb, GQ)], isl[j])
      pltpu.sync_copy(num2.at[pl.ds(eb, GQ * SZ), pl.ds(0, 16)], vsl[j])
      for jj in range(GQ):
        pltpu.async_copy(vsl[j].at[pl.ds(jj * SZ, SZ)],
                         acc.at[isl[j].at[jj]], ssem[j], add=True)
    return 0

  lax.fori_loop(0, NG_SCAT // 2, abody, 0)
  for j in range(2):
    pltpu.make_async_copy(
        num2.at[pl.ds(cbase * SZ, GQ * SZ), pl.ds(0, 16)], vsl[j],
        ssem[j]).wait()
  plsc.subcore_barrier()

  gbase = wid * NSC_W
  pltpu.sync_copy(src2.at[pl.ds(gbase, GQ)], is0)
  for jj in range(GQ):
    pltpu.async_copy(acc.at[is0.at[jj]], db0.at[pl.ds(jj * SZ, SZ)], gs0)

  def gbody(t, _):
    for j in range(2):
      gg = 2 * t + j
      cb = gbase + gg * GQ
      @pl.when(gg + 1 < NG_GATH)
      def _():
        pltpu.sync_copy(src2.at[pl.ds(cb + GQ, GQ)], isl[1 - j])
        for jj in range(GQ):
          pltpu.async_copy(acc.at[isl[1 - j].at[jj]],
                           dbl[1 - j].at[pl.ds(jj * SZ, SZ)], gsem[1 - j])
      pltpu.make_async_copy(
          num2.at[pl.ds(cb * SZ, GQ * SZ), pl.ds(0, 16)], dbl[j],
          gsem[j]).wait()
      pltpu.sync_copy(dbl[j],
                      dens_out.at[pl.ds(cb * SZ, GQ * SZ), pl.ds(0, 16)])
    return 0

  lax.fori_loop(0, NG_GATH // 2, gbody, 0)

  for j in range((NZN + NS - 1) // NS):
    ci = sid + j * NS
    @pl.when(jnp.logical_and(ci < NZN, (ci % NC) == cid))
    def _():
      pltpu.sync_copy(acc.at[pl.ds(ci * NZC, NZC)],
                      den_out.at[pl.ds(ci * NZC, NZC), pl.ds(0, 16)])


def _den_kernel(num2, src2):
  f = pl.kernel(
      _den_body,
      out_type=[jax.ShapeDtypeStruct((E, D), jnp.float32),
                jax.ShapeDtypeStruct((N, D), jnp.float32)],
      mesh=_mesh,
      compiler_params=_sc_params,
      scratch_types=[
          pltpu.VMEM((GQ, SZ), jnp.int32),
          pltpu.VMEM((GQ, SZ), jnp.int32),
          pltpu.VMEM((GQ * SZ, 16), jnp.float32),
          pltpu.VMEM((GQ * SZ, 16), jnp.float32),
          pltpu.VMEM((GQ * SZ, 16), jnp.float32),
          pltpu.VMEM((GQ * SZ, 16), jnp.float32),
          pltpu.VMEM((NZC, 16), jnp.float32),
          pltpu.VMEM_SHARED((N, 16), jnp.float32),
          pltpu.SemaphoreType.DMA,
          pltpu.SemaphoreType.DMA,
          pltpu.SemaphoreType.DMA,
          pltpu.SemaphoreType.DMA,
      ],
  )
  return f(num2, src2)


GQA = 4
NG_AGG = NSC_T // GQA


def _segsum_split_body(vals2, src2, out_hbm, is0, is1, vs0, vs1, zb, acc,
                       s0, s1):
  cid = lax.axis_index("c")
  sid = lax.axis_index("s")
  c0 = cid * W2
  isl = (is0, is1)
  vsl = (vs0, vs1)
  ssem = (s0, s1)

  zb[...] = jnp.zeros(zb.shape, jnp.float32)
  for j in range((NZN + NS - 1) // NS):
    ci = sid + j * NS
    @pl.when(ci < NZN)
    def _():
      pltpu.sync_copy(zb, acc.at[pl.ds(ci * NZC, NZC)])
  plsc.subcore_barrier()

  cbase = sid * NSC_T

  def abody(t, _):
    for j in range(2):
      gg = 2 * t + j
      cb = cbase + gg * GQA
      eb = cb * SZ
      @pl.when(t > 0)
      def _():
        pltpu.make_async_copy(
            vals2.at[pl.ds(eb, GQA * SZ), pl.ds(c0, W2)], vsl[j],
            ssem[j]).wait()
      pltpu.sync_copy(src2.at[pl.ds(cb, GQA)], isl[j])
      pltpu.sync_copy(vals2.at[pl.ds(eb, GQA * SZ), pl.ds(c0, W2)], vsl[j])
      for jj in range(GQA):
        pltpu.async_copy(vsl[j].at[pl.ds(jj * SZ, SZ)],
                         acc.at[isl[j].at[jj]], ssem[j], add=True)
    return 0

  lax.fori_loop(0, NG_AGG // 2, abody, 0)
  for j in range(2):
    pltpu.make_async_copy(
        vals2.at[pl.ds(cbase * SZ, GQA * SZ), pl.ds(c0, W2)], vsl[j],
        ssem[j]).wait()
  plsc.subcore_barrier()
  for j in range((NZN + NS - 1) // NS):
    ci = sid + j * NS
    @pl.when(ci < NZN)
    def _():
      pltpu.sync_copy(acc.at[pl.ds(ci * NZC, NZC)],
                      out_hbm.at[pl.ds(ci * NZC, NZC), pl.ds(c0, W2)])


def _segsum_split(vals2, src2):
  f = pl.kernel(
      _segsum_split_body,
      out_type=jax.ShapeDtypeStruct((N, D), jnp.float32),
      mesh=_mesh,
      compiler_params=_sc_params,
      scratch_types=[
          pltpu.VMEM((GQA, SZ), jnp.int32),
          pltpu.VMEM((GQA, SZ), jnp.int32),
          pltpu.VMEM((GQA * SZ, W2), jnp.float32),
          pltpu.VMEM((GQA * SZ, W2), jnp.float32),
          pltpu.VMEM((NZC, W2), jnp.float32),
          pltpu.VMEM_SHARED((N, W2), jnp.float32),
          pltpu.SemaphoreType.DMA,
          pltpu.SemaphoreType.DMA,
      ],
  )
  return f(vals2, src2)


def _attn_ffn_body(num_ref, den_ref, e_ref, weo_ref, beo_ref,
                   g_ref, b_ref, w1_ref, b1_ref, w2_ref, b2_ref,
                   attn_ref, oe_ref):
  attn128 = num_ref[...] / (den_ref[...] + 1e-12)
  attn_ref[...] = attn128
  eau = (jnp.dot(attn128, weo_ref[...],
                 preferred_element_type=jnp.float32) + beo_ref[...])
  es = e_ref[...] + eau
  x = _ln_block(es, g_ref[...], b_ref[...])
  h1 = jax.nn.relu(
      jnp.dot(x, w1_ref[...], preferred_element_type=jnp.float32) + b1_ref[...])
  ef = (jnp.dot(h1, w2_ref[...], preferred_element_type=jnp.float32)
        + b2_ref[...])
  oe_ref[...] = es + ef


def _attn_ffn(num, den_s, e, weo, beo, g, b, w1, b1, w2, b2):
  blk = 8000
  grid = E // blk
  row = pl.BlockSpec((blk, D), lambda i: (i, 0))
  n16 = row
  vec = pl.BlockSpec((D,), lambda i: (0,))
  return pl.pallas_call(
      _attn_ffn_body,
      grid=(grid,),
      in_specs=[n16, n16, row,
                pl.BlockSpec((D, D), lambda i: (0, 0)), vec,
                vec, vec,
                pl.BlockSpec((D, 2 * D), lambda i: (0, 0)),
                pl.BlockSpec((2 * D,), lambda i: (0,)),
                pl.BlockSpec((2 * D, D), lambda i: (0, 0)), vec],
      out_specs=[row, row],
      out_shape=[jax.ShapeDtypeStruct((E, D), jnp.float32),
                 jax.ShapeDtypeStruct((E, D), jnp.float32)],
  )(num, den_s, e, weo, beo, g, b, w1, b1, w2, b2)


def _node_post_body(u_ref, den_ref, x_ref, wno_ref, bno_ref, g_ref, b_ref,
                    w1_ref, b1_ref, w2_ref, b2_ref, o_ref):
  den_b = jnp.dot(den_ref[...][:, :16], _head_sel().T,
                  preferred_element_type=jnp.float32)
  agg = u_ref[...] / (den_b + 1e-30)
  nau = (jnp.dot(agg, wno_ref[...], preferred_element_type=jnp.float32)
         + bno_ref[...])
  ns = x_ref[...] + nau
  x = _ln_block(ns, g_ref[...], b_ref[...])
  h1 = jax.nn.relu(
      jnp.dot(x, w1_ref[...], preferred_element_type=jnp.float32) + b1_ref[...])
  nf = (jnp.dot(h1, w2_ref[...], preferred_element_type=jnp.float32)
        + b2_ref[...])
  o_ref[...] = ns + nf


def _node_post(u, den, x, wno, bno, g, b, w1, b1, w2, b2):
  blk = 1000
  grid = N // blk
  row = pl.BlockSpec((blk, D), lambda i: (i, 0))
  full = pl.BlockSpec((D, D), lambda i: (0, 0))
  vec = pl.BlockSpec((D,), lambda i: (0,))
  return pl.pallas_call(
      _node_post_body,
      grid=(grid,),
      in_specs=[row, row,
                row, full, vec, vec, vec,
                pl.BlockSpec((D, 2 * D), lambda i: (0, 0)),
                pl.BlockSpec((2 * D,), lambda i: (0,)),
                pl.BlockSpec((2 * D, D), lambda i: (0, 0)), vec],
      out_specs=row,
      out_shape=jax.ShapeDtypeStruct((N, D), jnp.float32),
  )(u, den, x, wno, bno, g, b, w1, b1, w2, b2)


@jax.jit
def kernel(node_states, edge_index, edge_states, params):
  p = params
  src = edge_index[0]
  dst = edge_index[1]

  q, kv = _node_pre(node_states, p['nln1_g'], p['nln1_b'],
                    p['wq'], p['bq'], p['wk'], p['bk'], p['wv'], p['bv'])

  ep = _edge_ep(edge_states, p['eln1_g'], p['eln1_b'], p['wes'], p['bes'])

  qs, kvd = _gather2(q, kv, src, dst)

  num3, uw3 = _edge_numuw(qs, kvd, ep)

  src2 = src.reshape(E // SZ, SZ)
  dens3, den = _den_kernel(num3, src2)

  weo_pad = jnp.pad(p['weo'], ((0, D - H), (0, 0)))
  attn128, out_edges = _attn_ffn(num3, dens3, edge_states, weo_pad, p['beo'],
                                 p['eln2_g'], p['eln2_b'],
                                 p['ef1_w'], p['ef1_b'],
                                 p['ef2_w'], p['ef2_b'])
  attn = attn128[:, :H]

  uagg = _segsum_split(uw3, src2)

  out_nodes = _node_post(uagg, den, node_states, p['wno'], p['bno'],
                         p['nln2_g'], p['nln2_b'],
                         p['nf1_w'], p['nf1_b'], p['nf2_w'], p['nf2_b'])

  return (out_nodes, out_edges, attn)
